# Initial kernel scaffold; baseline (speedup 1.0000x reference)
import jax
import jax.numpy as jnp
from jax.experimental import pallas as pl


def kernel(x, edge_index, W_in, b_in, l0_w1, l0_b1, l0_w2, l0_b2, l0_g, l0_be, l1_w1, l1_b1, l1_w2, l1_b2, l1_g, l1_be, l2_w1, l2_b1, l2_w2, l2_b2, l2_g, l2_be, out_w1, out_b1, out_w2, out_b2):
    raise NotImplementedError("write your pallas kernel here")


# speedup vs baseline: 1.8578x; 1.8578x over previous
import functools

import jax
import jax.numpy as jnp
from jax import lax
from jax.experimental import pallas as pl
from jax.experimental.pallas import tpu as pltpu
from jax.experimental.pallas import tpu_sc as plsc

N = 50000
E = 800000
H = 64
D_LAT = 32

NW = 32
R = 1568
NP = NW * R
EP = 802816
CAP = 28672
NE2 = NW * CAP // 2
GCH = 256
SCH = 256

_f32 = jnp.float32
_i32 = jnp.int32


BLK = 3136


def _rowmask_at(blk, i):
    it = lax.broadcasted_iota(_i32, (blk, 1), 0) + i * blk
    return it < N


def _dense0_body(x_ref, win_ref, bin_ref, wcat_ref, bcat_ref, h_ref, t_ref):
    i = pl.program_id(0)
    h = jnp.dot(x_ref[...], win_ref[...], preferred_element_type=_f32) + bin_ref[...]
    h = jnp.where(_rowmask_at(BLK, i), h, 0.0)
    h_ref[...] = h
    t_ref[...] = jnp.dot(h, wcat_ref[...], preferred_element_type=_f32) + bcat_ref[...]


def _tc_dense0(x_p, w_in, b_in, wcat, bcat):
    return pl.pallas_call(
        _dense0_body,
        grid=(NP // BLK,),
        in_specs=[
            pl.BlockSpec((BLK, 3), lambda i: (i, 0)),
            pl.BlockSpec((3, H), lambda i: (0, 0)),
            pl.BlockSpec((1, H), lambda i: (0, 0)),
            pl.BlockSpec((H, 2 * H), lambda i: (0, 0)),
            pl.BlockSpec((1, 2 * H), lambda i: (0, 0)),
        ],
        out_specs=(
            pl.BlockSpec((BLK, H), lambda i: (i, 0)),
            pl.BlockSpec((BLK, 2 * H), lambda i: (i, 0)),
        ),
        out_shape=(
            jax.ShapeDtypeStruct((NP, H), _f32),
            jax.ShapeDtypeStruct((NP, 2 * H), _f32),
        ),
    )(x_p, w_in, b_in.reshape(1, H), wcat, bcat)


def _edge_mlp_body(pre_ref, w2_ref, b2_ref, m_ref):
    a = jnp.maximum(pre_ref[:, :H], 0.0)
    b = jnp.maximum(pre_ref[:, H:], 0.0)
    ma = jnp.dot(a, w2_ref[...], preferred_element_type=_f32) + b2_ref[...]
    mb = jnp.dot(b, w2_ref[...], preferred_element_type=_f32) + b2_ref[...]
    m_ref[...] = jnp.concatenate([ma, mb], axis=1)


def _tc_edge_mlp(pre2, w2, b2):
    blk = 1024
    grid = NE2 // blk
    return pl.pallas_call(
        _edge_mlp_body,
        grid=(grid,),
        in_specs=[
            pl.BlockSpec((blk, 2 * H), lambda i: (i, 0)),
            pl.BlockSpec((H, H), lambda i: (0, 0)),
            pl.BlockSpec((1, H), lambda i: (0, 0)),
        ],
        out_specs=pl.BlockSpec((blk, 2 * H), lambda i: (i, 0)),
        out_shape=jax.ShapeDtypeStruct((NE2, 2 * H), _f32),
    )(pre2, w2, b2.reshape(1, H))


def _stats_body(first, h_ref, agg_ref, hp_ref, ssum_ref, ssq_ref):
    i = pl.program_id(0)
    aggc = jnp.where(jnp.isneginf(agg_ref[...]), 0.0, agg_ref[...])
    aggc = jnp.where(_rowmask_at(BLK, i), aggc, 0.0)
    hp = aggc if first else h_ref[...] + aggc
    hp_ref[...] = hp

    @pl.when(i == 0)
    def _():
        ssum_ref[...] = jnp.zeros((1, H), _f32)
        ssq_ref[...] = jnp.zeros((1, H), _f32)
    ssum_ref[...] += jnp.sum(hp, axis=0, keepdims=True)
    ssq_ref[...] += jnp.sum(hp * hp, axis=0, keepdims=True)


def _tc_stats(h, agg, first):
    return pl.pallas_call(
        functools.partial(_stats_body, first),
        grid=(NP // BLK,),
        in_specs=[
            pl.BlockSpec((BLK, H), lambda i: (i, 0)),
            pl.BlockSpec((BLK, H), lambda i: (i, 0)),
        ],
        out_specs=(
            pl.BlockSpec((BLK, H), lambda i: (i, 0)),
            pl.BlockSpec((1, H), lambda i: (0, 0)),
            pl.BlockSpec((1, H), lambda i: (0, 0)),
        ),
        out_shape=(
            jax.ShapeDtypeStruct((NP, H), _f32),
            jax.ShapeDtypeStruct((1, H), _f32),
            jax.ShapeDtypeStruct((1, H), _f32),
        ),
    )(h, agg)


def _hn_block(hp, ssum, ssq, g, be, i):
    mean = ssum / N
    var = ssq / N - mean * mean
    hn = g * (hp - mean) * lax.rsqrt(var + 1e-5) + be
    hn = jnp.maximum(hn, 0.0)
    return jnp.where(_rowmask_at(BLK, i), hn, 0.0)


def _apply_body(hp_ref, ssum_ref, ssq_ref, g_ref, be_ref,
                wcat_ref, bcat_ref, h_ref, t_ref):
    i = pl.program_id(0)
    hn = _hn_block(hp_ref[...], ssum_ref[...], ssq_ref[...],
                   g_ref[...], be_ref[...], i)
    h_ref[...] = hn
    t_ref[...] = jnp.dot(hn, wcat_ref[...], preferred_element_type=_f32) + bcat_ref[...]


def _tc_apply(hp, ssum, ssq, g, be, wcat, bcat):
    return pl.pallas_call(
        _apply_body,
        grid=(NP // BLK,),
        in_specs=[
            pl.BlockSpec((BLK, H), lambda i: (i, 0)),
            pl.BlockSpec((1, H), lambda i: (0, 0)),
            pl.BlockSpec((1, H), lambda i: (0, 0)),
            pl.BlockSpec((1, H), lambda i: (0, 0)),
            pl.BlockSpec((1, H), lambda i: (0, 0)),
            pl.BlockSpec((H, 2 * H), lambda i: (0, 0)),
            pl.BlockSpec((1, 2 * H), lambda i: (0, 0)),
        ],
        out_specs=(
            pl.BlockSpec((BLK, H), lambda i: (i, 0)),
            pl.BlockSpec((BLK, 2 * H), lambda i: (i, 0)),
        ),
        out_shape=(
            jax.ShapeDtypeStruct((NP, H), _f32),
            jax.ShapeDtypeStruct((NP, 2 * H), _f32),
        ),
    )(hp, ssum, ssq, g.reshape(1, H), be.reshape(1, H), wcat, bcat)


def _final_sum_body(hp_ref, ssum_ref, ssq_ref, g_ref, be_ref, zsum_ref):
    i = pl.program_id(0)
    hn = _hn_block(hp_ref[...], ssum_ref[...], ssq_ref[...],
                   g_ref[...], be_ref[...], i)

    @pl.when(i == 0)
    def _():
        zsum_ref[...] = jnp.zeros((1, H), _f32)
    zsum_ref[...] += jnp.sum(hn, axis=0, keepdims=True)


def _head_body(zsum_ref, ow1_ref, ob1_ref, ow2_ref, ob2_ref, z_ref):
    z = zsum_ref[...] / N
    z = jnp.maximum(jnp.dot(z, ow1_ref[...], preferred_element_type=_f32)
                    + ob1_ref[...], 0.0)
    z_ref[...] = jnp.dot(z, ow2_ref[...], preferred_element_type=_f32) + ob2_ref[...]


def _tc_final(hp, ssum, ssq, g, be, ow1, ob1, ow2, ob2):
    zsum = pl.pallas_call(
        _final_sum_body,
        grid=(NP // BLK,),
        in_specs=[
            pl.BlockSpec((BLK, H), lambda i: (i, 0)),
            pl.BlockSpec((1, H), lambda i: (0, 0)),
            pl.BlockSpec((1, H), lambda i: (0, 0)),
            pl.BlockSpec((1, H), lambda i: (0, 0)),
            pl.BlockSpec((1, H), lambda i: (0, 0)),
        ],
        out_specs=pl.BlockSpec((1, H), lambda i: (0, 0)),
        out_shape=jax.ShapeDtypeStruct((1, H), _f32),
    )(hp, ssum, ssq, g.reshape(1, H), be.reshape(1, H))
    return pl.pallas_call(
        _head_body,
        out_shape=jax.ShapeDtypeStruct((1, D_LAT), _f32),
    )(zsum, ow1, ob1.reshape(1, H), ow2, ob2.reshape(1, D_LAT))


@functools.lru_cache(maxsize=None)
def _mesh():
    return plsc.VectorSubcoreMesh(core_axis_name="c", subcore_axis_name="s")


def _wid():
    return lax.axis_index("s") * 2 + lax.axis_index("c")


def _sc_prep(dst_p, src_p):
    @functools.partial(
        pl.kernel, mesh=_mesh(),
        compiler_params=pltpu.CompilerParams(needs_layout_passes=False),
        out_type=[
            jax.ShapeDtypeStruct((NW * CAP,), _i32),
            jax.ShapeDtypeStruct((NW * CAP,), _i32),
            jax.ShapeDtypeStruct((NW * 128,), _i32),
        ],
        scratch_types=[
            pltpu.VMEM((CAP,), _i32),
            pltpu.VMEM((CAP,), _i32),
            pltpu.VMEM((2048,), _i32),
            pltpu.VMEM((2048,), _i32),
            pltpu.VMEM((128,), _i32),
        ],
    )
    def k(dst_hbm, src_hbm, dloc_hbm, bsrc_hbm, cnt_hbm,
          dloc_v, bsrc_v, dchunk_v, schunk_v, cnt_v):
        w = _wid()
        base_lo = w * R
        lanes = lax.iota(_i32, 16)

        def fill(i, _):
            dloc_v[pl.ds(i * 16, 16)] = jnp.full((16,), R, _i32)
            bsrc_v[pl.ds(i * 16, 16)] = i * 16 + lanes
            return 0
        lax.fori_loop(0, CAP // 16, fill, 0)

        def chunk(c, off):
            cb = pl.multiple_of(c * 2048, 2048)
            pltpu.sync_copy(dst_hbm.at[pl.ds(cb, 2048)], dchunk_v)
            pltpu.sync_copy(src_hbm.at[pl.ds(cb, 2048)], schunk_v)

            def group(gidx, off):
                dl = dchunk_v[pl.ds(gidx * 16, 16)] - base_lo
                sv = schunk_v[pl.ds(gidx * 16, 16)]
                mask = (dl >= 0) & (dl < R)
                cs = plsc.cumsum(mask.astype(_i32))
                pos = jnp.minimum(off + cs - 1, CAP - 1)
                plsc.store_scatter(dloc_v, [pos], dl, mask=mask)
                plsc.store_scatter(bsrc_v, [pos], sv, mask=mask)
                return off + jnp.max(cs, axis=0)
            return lax.fori_loop(0, 128, group, off)

        off = lax.fori_loop(0, EP // 2048, chunk, jnp.asarray(0, _i32))
        off = jnp.minimum(off, CAP - 16)
        wb = pl.multiple_of(w * CAP, CAP)
        pltpu.sync_copy(dloc_v, dloc_hbm.at[pl.ds(wb, CAP)])
        pltpu.sync_copy(bsrc_v, bsrc_hbm.at[pl.ds(wb, CAP)])

        def cfill(i, _):
            cnt_v[pl.ds(i * 16, 16)] = jnp.full((16,), off, _i32)
            return 0
        lax.fori_loop(0, 8, cfill, 0)
        pltpu.sync_copy(cnt_v, cnt_hbm.at[pl.ds(pl.multiple_of(w * 128, 128), 128)])

    return k(dst_p, src_p)


def _sc_gather(t_tab, dloc, bsrc, cnt):
    @functools.partial(
        pl.kernel, mesh=_mesh(),
        compiler_params=pltpu.CompilerParams(needs_layout_passes=False),
        out_type=jax.ShapeDtypeStruct((NE2, 2 * H), _f32),
        scratch_types=[
            pltpu.VMEM((GCH,), _i32),
            pltpu.VMEM((GCH,), _i32),
            pltpu.VMEM((GCH, 2 * H), _f32),
            pltpu.VMEM((GCH, 2 * H), _f32),
            pltpu.VMEM((GCH // 2, 2 * H), _f32),
            pltpu.SemaphoreType.DMA,
        ],
    )
    def k(t_hbm, dloc_hbm, bsrc_hbm, cnt_hbm, pre_hbm,
          idxd_v, idxs_v, bufd_v, bufs_v, pre_v, sem):
        w = _wid()
        wr = w * R

        def chunk(c, _):
            cb = pl.multiple_of(w * CAP + c * GCH, GCH)
            pltpu.sync_copy(dloc_hbm.at[pl.ds(cb, GCH)], idxd_v)
            pltpu.sync_copy(bsrc_hbm.at[pl.ds(cb, GCH)], idxs_v)

            def toglobal(i, _):
                s = pl.ds(i * 16, 16)
                idxd_v[s] = jnp.minimum(idxd_v[s] + wr, NP - 1)
                return 0
            lax.fori_loop(0, GCH // 16, toglobal, 0)

            cps = []
            for j in range(GCH // 128):
                s = pl.ds(j * 128, 128)
                cps.append(pltpu.async_copy(
                    t_hbm.at[idxd_v.at[s]], bufd_v.at[s], sem))
                cps.append(pltpu.async_copy(
                    t_hbm.at[idxs_v.at[s]], bufs_v.at[s], sem))
            for cp in cps:
                cp.wait()

            def combine(i, _):
                for e2 in range(2):
                    e = 2 * i + e2
                    for q in range(H // 16):
                        a = bufd_v[e, pl.ds(q * 16, 16)]
                        b = bufs_v[e, pl.ds(H + q * 16, 16)]
                        pre_v[i, pl.ds(e2 * H + q * 16, 16)] = a + b
                return 0
            lax.fori_loop(0, GCH // 2, combine, 0)

            pltpu.sync_copy(
                pre_v, pre_hbm.at[pl.ds(pl.multiple_of(cb // 2, GCH // 2), GCH // 2)])
            return 0

        lax.fori_loop(0, CAP // GCH, chunk, 0)

    return k(t_tab, dloc, bsrc, cnt)


def _sc_scatter_max(m2, dloc, cnt):
    @functools.partial(
        pl.kernel, mesh=_mesh(),
        compiler_params=pltpu.CompilerParams(needs_layout_passes=False),
        out_type=jax.ShapeDtypeStruct((NP // 2, 2 * H), _f32),
        scratch_types=[
            pltpu.VMEM((R // 2 + 1, 2 * H), _f32),
            pltpu.VMEM((SCH // 2, 2 * H), _f32),
            pltpu.VMEM((SCH,), _i32),
            pltpu.VMEM((128,), _i32),
            pltpu.SemaphoreType.DMA,
        ],
    )
    def k(m_hbm, dloc_hbm, cnt_hbm, agg_hbm, acc_v, rows_v, dloc_v, cnt_v, sem):
        w = _wid()
        lanes = lax.iota(_i32, 16)
        neg = jnp.full((16,), -jnp.inf, _f32)

        def fill(i, _):
            for q in range(2 * H // 16):
                acc_v[i, pl.ds(q * 16, 16)] = neg
            return 0
        lax.fori_loop(0, R // 2 + 1, fill, 0)

        pltpu.sync_copy(cnt_hbm.at[pl.ds(pl.multiple_of(w * 128, 128), 128)], cnt_v)
        cnt = jnp.max(cnt_v[pl.ds(0, 16)], axis=0)
        nch = (cnt + (SCH - 1)) // SCH

        def chunk(c, _):
            cb = pl.multiple_of(w * CAP + c * SCH, SCH)
            pltpu.sync_copy(dloc_hbm.at[pl.ds(cb, SCH)], dloc_v)
            pltpu.sync_copy(
                m_hbm.at[pl.ds(pl.multiple_of(cb // 2, SCH // 2), SCH // 2)], rows_v)

            def group(gidx, _):
                dl = dloc_v[pl.ds(gidx * 16, 16)]
                for e in range(16):
                    j = gidx * 16 + e
                    d = jnp.max(jnp.where(lanes == e, dl, 0), axis=0)
                    drow = d // 2
                    dcol = (d % 2) * H
                    for q in range(H // 16):
                        cur = acc_v[drow, pl.ds(dcol + q * 16, 16)]
                        val = rows_v[j // 2, pl.ds((j % 2) * H + q * 16, 16)]
                        acc_v[drow, pl.ds(dcol + q * 16, 16)] = jnp.maximum(cur, val)
                return 0
            return lax.fori_loop(0, SCH // 16, group, 0)

        lax.fori_loop(0, nch, chunk, 0)
        pltpu.sync_copy(acc_v.at[pl.ds(0, R // 2)],
                        agg_hbm.at[pl.ds(pl.multiple_of(w * (R // 2), R // 2), R // 2)])

    return k(m2, dloc, cnt)


def kernel(x, edge_index, W_in, b_in,
           l0_w1, l0_b1, l0_w2, l0_b2, l0_g, l0_be,
           l1_w1, l1_b1, l1_w2, l1_b2, l1_g, l1_be,
           l2_w1, l2_b1, l2_w2, l2_b2, l2_g, l2_be,
           out_w1, out_b1, out_w2, out_b2):
    x_p = jnp.pad(x, ((0, NP - N), (0, 0)))
    pad_idx = N + (jnp.arange(EP - E, dtype=_i32) % (NP - N))
    dst_p = jnp.concatenate([edge_index[1], pad_idx])
    src_p = jnp.concatenate([edge_index[0], pad_idx % N])

    layers = [(l0_w1, l0_b1, l0_w2, l0_b2, l0_g, l0_be),
              (l1_w1, l1_b1, l1_w2, l1_b2, l1_g, l1_be),
              (l2_w1, l2_b1, l2_w2, l2_b2, l2_g, l2_be)]
    wcat = [jnp.concatenate([w1[:H] - w1[H:], w1[H:]], axis=1)
            for (w1, _, _, _, _, _) in layers]
    bcat = [jnp.concatenate([b1, jnp.zeros((H,), _f32)]).reshape(1, 2 * H)
            for (_, b1, _, _, _, _) in layers]

    dloc, bsrc, cnt = _sc_prep(dst_p, src_p)
    h, t = _tc_dense0(x_p, W_in, b_in, wcat[0], bcat[0])

    for i, (_, b1, w2, b2, g, be) in enumerate(layers):
        pre2 = _sc_gather(t, dloc, bsrc, cnt)
        m2 = _tc_edge_mlp(pre2, w2, b2)
        agg2 = _sc_scatter_max(m2, dloc, cnt)
        agg = jnp.reshape(agg2, (NP, H))
        hp, ssum, ssq = _tc_stats(h, agg, first=(i == 0))
        if i < 2:
            h, t = _tc_apply(hp, ssum, ssq, g, be, wcat[i + 1], bcat[i + 1])
        else:
            z = _tc_final(hp, ssum, ssq, g, be, out_w1, out_b1, out_w2, out_b2)
    return z


# speedup vs baseline: 2.4267x; 1.3062x over previous
import functools

import jax
import jax.numpy as jnp
from jax import lax
from jax.experimental import pallas as pl
from jax.experimental.pallas import tpu as pltpu
from jax.experimental.pallas import tpu_sc as plsc

N = 50000
E = 800000
H = 64
D_LAT = 32

NW = 32
R = 1568
NP = NW * R
EP = 802816
CAP = 28672
NE2 = NW * CAP // 2
GCH = 64
SUP = 1024
SCH = 256

_f32 = jnp.float32
_i32 = jnp.int32


BLK = 3136


def _rowmask_at(blk, i):
    it = lax.broadcasted_iota(_i32, (blk, 1), 0) + i * blk
    return it < N


def _dense0_body(x_ref, win_ref, bin_ref, wcat_ref, bcat_ref, h_ref, t_ref, p_ref):
    i = pl.program_id(0)
    h = jnp.dot(x_ref[...], win_ref[...], preferred_element_type=_f32) + bin_ref[...]
    h = jnp.where(_rowmask_at(BLK, i), h, 0.0)
    h_ref[...] = h
    t = jnp.dot(h, wcat_ref[...], preferred_element_type=_f32) + bcat_ref[...]
    t_ref[...] = t
    p_ref[...] = t[:, H:]


def _tc_dense0(x_p, w_in, b_in, wcat, bcat):
    return pl.pallas_call(
        _dense0_body,
        grid=(NP // BLK,),
        in_specs=[
            pl.BlockSpec((BLK, 3), lambda i: (i, 0)),
            pl.BlockSpec((3, H), lambda i: (0, 0)),
            pl.BlockSpec((1, H), lambda i: (0, 0)),
            pl.BlockSpec((H, 2 * H), lambda i: (0, 0)),
            pl.BlockSpec((1, 2 * H), lambda i: (0, 0)),
        ],
        out_specs=(
            pl.BlockSpec((BLK, H), lambda i: (i, 0)),
            pl.BlockSpec((BLK, 2 * H), lambda i: (i, 0)),
            pl.BlockSpec((BLK, H), lambda i: (i, 0)),
        ),
        out_shape=(
            jax.ShapeDtypeStruct((NP, H), _f32),
            jax.ShapeDtypeStruct((NP, 2 * H), _f32),
            jax.ShapeDtypeStruct((NP, H), _f32),
        ),
    )(x_p, w_in, b_in.reshape(1, H), wcat, bcat)


def _edge_mlp_body(pre_ref, w2_ref, b2_ref, m_ref):
    a = jnp.maximum(pre_ref[:, :H], 0.0)
    b = jnp.maximum(pre_ref[:, H:], 0.0)
    ma = jnp.dot(a, w2_ref[...], preferred_element_type=_f32) + b2_ref[...]
    mb = jnp.dot(b, w2_ref[...], preferred_element_type=_f32) + b2_ref[...]
    m_ref[...] = jnp.concatenate([ma, mb], axis=1)


def _tc_edge_mlp(pre2, w2, b2):
    blk = 1024
    grid = NE2 // blk
    return pl.pallas_call(
        _edge_mlp_body,
        grid=(grid,),
        in_specs=[
            pl.BlockSpec((blk, 2 * H), lambda i: (i, 0)),
            pl.BlockSpec((H, H), lambda i: (0, 0)),
            pl.BlockSpec((1, H), lambda i: (0, 0)),
        ],
        out_specs=pl.BlockSpec((blk, 2 * H), lambda i: (i, 0)),
        out_shape=jax.ShapeDtypeStruct((NE2, 2 * H), _f32),
    )(pre2, w2, b2.reshape(1, H))


def _stats_body(first, h_ref, agg_ref, hp_ref, ssum_ref, ssq_ref):
    i = pl.program_id(0)
    aggc = jnp.where(jnp.isneginf(agg_ref[...]), 0.0, agg_ref[...])
    aggc = jnp.where(_rowmask_at(BLK, i), aggc, 0.0)
    hp = aggc if first else h_ref[...] + aggc
    hp_ref[...] = hp

    @pl.when(i == 0)
    def _():
        ssum_ref[...] = jnp.zeros((1, H), _f32)
        ssq_ref[...] = jnp.zeros((1, H), _f32)
    ssum_ref[...] += jnp.sum(hp, axis=0, keepdims=True)
    ssq_ref[...] += jnp.sum(hp * hp, axis=0, keepdims=True)


def _tc_stats(h, agg, first):
    return pl.pallas_call(
        functools.partial(_stats_body, first),
        grid=(NP // BLK,),
        in_specs=[
            pl.BlockSpec((BLK, H), lambda i: (i, 0)),
            pl.BlockSpec((BLK, H), lambda i: (i, 0)),
        ],
        out_specs=(
            pl.BlockSpec((BLK, H), lambda i: (i, 0)),
            pl.BlockSpec((1, H), lambda i: (0, 0)),
            pl.BlockSpec((1, H), lambda i: (0, 0)),
        ),
        out_shape=(
            jax.ShapeDtypeStruct((NP, H), _f32),
            jax.ShapeDtypeStruct((1, H), _f32),
            jax.ShapeDtypeStruct((1, H), _f32),
        ),
    )(h, agg)


def _hn_block(hp, ssum, ssq, g, be, i):
    mean = ssum / N
    var = ssq / N - mean * mean
    hn = g * (hp - mean) * lax.rsqrt(var + 1e-5) + be
    hn = jnp.maximum(hn, 0.0)
    return jnp.where(_rowmask_at(BLK, i), hn, 0.0)


def _apply_body(hp_ref, ssum_ref, ssq_ref, g_ref, be_ref,
                wcat_ref, bcat_ref, h_ref, t_ref, p_ref):
    i = pl.program_id(0)
    hn = _hn_block(hp_ref[...], ssum_ref[...], ssq_ref[...],
                   g_ref[...], be_ref[...], i)
    h_ref[...] = hn
    t = jnp.dot(hn, wcat_ref[...], preferred_element_type=_f32) + bcat_ref[...]
    t_ref[...] = t
    p_ref[...] = t[:, H:]


def _tc_apply(hp, ssum, ssq, g, be, wcat, bcat):
    return pl.pallas_call(
        _apply_body,
        grid=(NP // BLK,),
        in_specs=[
            pl.BlockSpec((BLK, H), lambda i: (i, 0)),
            pl.BlockSpec((1, H), lambda i: (0, 0)),
            pl.BlockSpec((1, H), lambda i: (0, 0)),
            pl.BlockSpec((1, H), lambda i: (0, 0)),
            pl.BlockSpec((1, H), lambda i: (0, 0)),
            pl.BlockSpec((H, 2 * H), lambda i: (0, 0)),
            pl.BlockSpec((1, 2 * H), lambda i: (0, 0)),
        ],
        out_specs=(
            pl.BlockSpec((BLK, H), lambda i: (i, 0)),
            pl.BlockSpec((BLK, 2 * H), lambda i: (i, 0)),
            pl.BlockSpec((BLK, H), lambda i: (i, 0)),
        ),
        out_shape=(
            jax.ShapeDtypeStruct((NP, H), _f32),
            jax.ShapeDtypeStruct((NP, 2 * H), _f32),
            jax.ShapeDtypeStruct((NP, H), _f32),
        ),
    )(hp, ssum, ssq, g.reshape(1, H), be.reshape(1, H), wcat, bcat)


def _final_sum_body(hp_ref, ssum_ref, ssq_ref, g_ref, be_ref, zsum_ref):
    i = pl.program_id(0)
    hn = _hn_block(hp_ref[...], ssum_ref[...], ssq_ref[...],
                   g_ref[...], be_ref[...], i)

    @pl.when(i == 0)
    def _():
        zsum_ref[...] = jnp.zeros((1, H), _f32)
    zsum_ref[...] += jnp.sum(hn, axis=0, keepdims=True)


def _head_body(zsum_ref, ow1_ref, ob1_ref, ow2_ref, ob2_ref, z_ref):
    z = zsum_ref[...] / N
    z = jnp.maximum(jnp.dot(z, ow1_ref[...], preferred_element_type=_f32)
                    + ob1_ref[...], 0.0)
    z_ref[...] = jnp.dot(z, ow2_ref[...], preferred_element_type=_f32) + ob2_ref[...]


def _tc_final(hp, ssum, ssq, g, be, ow1, ob1, ow2, ob2):
    zsum = pl.pallas_call(
        _final_sum_body,
        grid=(NP // BLK,),
        in_specs=[
            pl.BlockSpec((BLK, H), lambda i: (i, 0)),
            pl.BlockSpec((1, H), lambda i: (0, 0)),
            pl.BlockSpec((1, H), lambda i: (0, 0)),
            pl.BlockSpec((1, H), lambda i: (0, 0)),
            pl.BlockSpec((1, H), lambda i: (0, 0)),
        ],
        out_specs=pl.BlockSpec((1, H), lambda i: (0, 0)),
        out_shape=jax.ShapeDtypeStruct((1, H), _f32),
    )(hp, ssum, ssq, g.reshape(1, H), be.reshape(1, H))
    return pl.pallas_call(
        _head_body,
        out_shape=jax.ShapeDtypeStruct((1, D_LAT), _f32),
    )(zsum, ow1, ob1.reshape(1, H), ow2, ob2.reshape(1, D_LAT))


@functools.lru_cache(maxsize=None)
def _mesh():
    return plsc.VectorSubcoreMesh(core_axis_name="c", subcore_axis_name="s")


def _wid():
    return lax.axis_index("s") * 2 + lax.axis_index("c")


def _sc_prep(dst_p, src_p):
    @functools.partial(
        pl.kernel, mesh=_mesh(),
        compiler_params=pltpu.CompilerParams(needs_layout_passes=False),
        out_type=[
            jax.ShapeDtypeStruct((NW * CAP,), _i32),
            jax.ShapeDtypeStruct((NW * CAP,), _i32),
            jax.ShapeDtypeStruct((NW * 128,), _i32),
        ],
        scratch_types=[
            pltpu.VMEM((CAP,), _i32),
            pltpu.VMEM((CAP,), _i32),
            pltpu.VMEM((2048,), _i32),
            pltpu.VMEM((2048,), _i32),
            pltpu.VMEM((128,), _i32),
        ],
    )
    def k(dst_hbm, src_hbm, dloc_hbm, bsrc_hbm, cnt_hbm,
          dloc_v, bsrc_v, dchunk_v, schunk_v, cnt_v):
        w = _wid()
        base_lo = w * R
        lanes = lax.iota(_i32, 16)

        def fill(i, _):
            dloc_v[pl.ds(i * 16, 16)] = jnp.full((16,), R, _i32)
            bsrc_v[pl.ds(i * 16, 16)] = i * 16 + lanes
            return 0
        lax.fori_loop(0, CAP // 16, fill, 0)

        def chunk(c, off):
            cb = pl.multiple_of(c * 2048, 2048)
            pltpu.sync_copy(dst_hbm.at[pl.ds(cb, 2048)], dchunk_v)
            pltpu.sync_copy(src_hbm.at[pl.ds(cb, 2048)], schunk_v)

            def group(gidx, off):
                dl = dchunk_v[pl.ds(gidx * 16, 16)] - base_lo
                sv = schunk_v[pl.ds(gidx * 16, 16)]
                mask = (dl >= 0) & (dl < R)
                cs = plsc.cumsum(mask.astype(_i32))
                pos = jnp.minimum(off + cs - 1, CAP - 1)
                plsc.store_scatter(dloc_v, [pos], dl, mask=mask)
                plsc.store_scatter(bsrc_v, [pos], sv, mask=mask)
                return off + jnp.max(cs, axis=0)
            return lax.fori_loop(0, 128, group, off)

        off = lax.fori_loop(0, EP // 2048, chunk, jnp.asarray(0, _i32))
        off = jnp.minimum(off, CAP - 16)
        wb = pl.multiple_of(w * CAP, CAP)
        pltpu.sync_copy(dloc_v, dloc_hbm.at[pl.ds(wb, CAP)])
        pltpu.sync_copy(bsrc_v, bsrc_hbm.at[pl.ds(wb, CAP)])

        def cfill(i, _):
            cnt_v[pl.ds(i * 16, 16)] = jnp.full((16,), off, _i32)
            return 0
        lax.fori_loop(0, 8, cfill, 0)
        pltpu.sync_copy(cnt_v, cnt_hbm.at[pl.ds(pl.multiple_of(w * 128, 128), 128)])

    return k(dst_p, src_p)


def _sc_gather(t_tab, p_tab, dloc, bsrc):
    @functools.partial(
        pl.kernel, mesh=_mesh(),
        compiler_params=pltpu.CompilerParams(needs_layout_passes=False),
        out_type=jax.ShapeDtypeStruct((NE2, 2 * H), _f32),
        scratch_types=[
            pltpu.VMEM(((R + 1) * H,), _f32),
            pltpu.VMEM((SUP,), _i32),
            pltpu.VMEM((SUP,), _i32),
            pltpu.VMEM((GCH, 2 * H), _f32),
            pltpu.VMEM((GCH, 2 * H), _f32),
            pltpu.VMEM((GCH // 2, 2 * H), _f32),
            pltpu.VMEM((GCH // 2, 2 * H), _f32),
            pltpu.SemaphoreType.DMA,
            pltpu.SemaphoreType.DMA,
            pltpu.SemaphoreType.DMA,
            pltpu.SemaphoreType.DMA,
        ],
    )
    def k(t_hbm, p_hbm, dloc_hbm, bsrc_hbm, pre_hbm,
          p_v, dl_v, sr_v, q0_v, q1_v, pre0_v, pre1_v, g0, g1, w0, w1):
        w = _wid()
        wr = pl.multiple_of(w * R, R)
        lanes = lax.iota(_i32, 16)
        qb = (q0_v, q1_v)
        pb = (pre0_v, pre1_v)
        gs = (g0, g1)
        ws = (w0, w1)
        ninner = SUP // GCH

        pltpu.sync_copy(p_hbm.at[pl.ds(pl.multiple_of(w * (R * H), R * H), R * H)],
                        p_v.at[pl.ds(0, R * H)])

        def sup(c, _):
            sb = pl.multiple_of(w * CAP + c * SUP, SUP)
            pltpu.sync_copy(dloc_hbm.at[pl.ds(sb, SUP)], dl_v)
            pltpu.sync_copy(bsrc_hbm.at[pl.ds(sb, SUP)], sr_v)
            for j in range(2):
                pltpu.async_copy(
                    t_hbm.at[sr_v.at[pl.ds(j * GCH, GCH)]], qb[j], gs[j])
            for j in range(ninner):
                b = j & 1
                pltpu.make_async_copy(
                    t_hbm.at[sr_v.at[pl.ds(j * GCH, GCH)]], qb[b], gs[b]).wait()
                if j >= 2:
                    pltpu.make_async_copy(
                        pb[b], pre_hbm.at[pl.ds(0, GCH // 2)], ws[b]).wait()

                def grp(gi, _):
                    dl = dl_v[pl.ds(j * GCH + gi * 16, 16)]
                    for e in range(16):
                        d = jnp.max(jnp.where(lanes == e, dl, 0), axis=0)
                        erow = (gi * 16 + e) // 2
                        ecol = ((gi * 16 + e) % 2) * H
                        for q in range(H // 16):
                            a = p_v[pl.ds(d * H + q * 16, 16)]
                            b_ = qb[b][gi * 16 + e, pl.ds(q * 16, 16)]
                            pb[b][erow, pl.ds(ecol + q * 16, 16)] = a + b_
                    return 0
                lax.fori_loop(0, GCH // 16, grp, 0)

                ob = pl.multiple_of((sb + j * GCH) // 2, GCH // 2)
                pltpu.async_copy(pb[b], pre_hbm.at[pl.ds(ob, GCH // 2)], ws[b])
                if j + 2 < ninner:
                    pltpu.async_copy(
                        t_hbm.at[sr_v.at[pl.ds((j + 2) * GCH, GCH)]], qb[b], gs[b])
            for j in range(2):
                pltpu.make_async_copy(
                    pb[j], pre_hbm.at[pl.ds(0, GCH // 2)], ws[j]).wait()
            return 0

        lax.fori_loop(0, CAP // SUP, sup, 0)

    return k(t_tab, p_tab, dloc, bsrc)


def _sc_scatter_max(m2, dloc, cnt):
    @functools.partial(
        pl.kernel, mesh=_mesh(),
        compiler_params=pltpu.CompilerParams(needs_layout_passes=False),
        out_type=jax.ShapeDtypeStruct((NP // 2, 2 * H), _f32),
        scratch_types=[
            pltpu.VMEM((R // 2 + 1, 2 * H), _f32),
            pltpu.VMEM((SCH // 2, 2 * H), _f32),
            pltpu.VMEM((SCH,), _i32),
            pltpu.VMEM((128,), _i32),
            pltpu.SemaphoreType.DMA,
        ],
    )
    def k(m_hbm, dloc_hbm, cnt_hbm, agg_hbm, acc_v, rows_v, dloc_v, cnt_v, sem):
        w = _wid()
        lanes = lax.iota(_i32, 16)
        neg = jnp.full((16,), -jnp.inf, _f32)

        def fill(i, _):
            for q in range(2 * H // 16):
                acc_v[i, pl.ds(q * 16, 16)] = neg
            return 0
        lax.fori_loop(0, R // 2 + 1, fill, 0)

        pltpu.sync_copy(cnt_hbm.at[pl.ds(pl.multiple_of(w * 128, 128), 128)], cnt_v)
        cnt = jnp.max(cnt_v[pl.ds(0, 16)], axis=0)
        nch = (cnt + (SCH - 1)) // SCH

        def chunk(c, _):
            cb = pl.multiple_of(w * CAP + c * SCH, SCH)
            pltpu.sync_copy(dloc_hbm.at[pl.ds(cb, SCH)], dloc_v)
            pltpu.sync_copy(
                m_hbm.at[pl.ds(pl.multiple_of(cb // 2, SCH // 2), SCH // 2)], rows_v)

            def group(gidx, _):
                dl = dloc_v[pl.ds(gidx * 16, 16)]
                for e in range(16):
                    j = gidx * 16 + e
                    d = jnp.max(jnp.where(lanes == e, dl, 0), axis=0)
                    drow = d // 2
                    dcol = (d % 2) * H
                    for q in range(H // 16):
                        cur = acc_v[drow, pl.ds(dcol + q * 16, 16)]
                        val = rows_v[j // 2, pl.ds((j % 2) * H + q * 16, 16)]
                        acc_v[drow, pl.ds(dcol + q * 16, 16)] = jnp.maximum(cur, val)
                return 0
            return lax.fori_loop(0, SCH // 16, group, 0)

        lax.fori_loop(0, nch, chunk, 0)
        pltpu.sync_copy(acc_v.at[pl.ds(0, R // 2)],
                        agg_hbm.at[pl.ds(pl.multiple_of(w * (R // 2), R // 2), R // 2)])

    return k(m2, dloc, cnt)


def kernel(x, edge_index, W_in, b_in,
           l0_w1, l0_b1, l0_w2, l0_b2, l0_g, l0_be,
           l1_w1, l1_b1, l1_w2, l1_b2, l1_g, l1_be,
           l2_w1, l2_b1, l2_w2, l2_b2, l2_g, l2_be,
           out_w1, out_b1, out_w2, out_b2):
    x_p = jnp.pad(x, ((0, NP - N), (0, 0)))
    pad_idx = N + (jnp.arange(EP - E, dtype=_i32) % (NP - N))
    dst_p = jnp.concatenate([edge_index[1], pad_idx])
    src_p = jnp.concatenate([edge_index[0], pad_idx % N])

    layers = [(l0_w1, l0_b1, l0_w2, l0_b2, l0_g, l0_be),
              (l1_w1, l1_b1, l1_w2, l1_b2, l1_g, l1_be),
              (l2_w1, l2_b1, l2_w2, l2_b2, l2_g, l2_be)]
    wcat = [jnp.concatenate([w1[H:], w1[:H] - w1[H:]], axis=1)
            for (w1, _, _, _, _, _) in layers]
    bcat = [jnp.concatenate([jnp.zeros((H,), _f32), b1]).reshape(1, 2 * H)
            for (_, b1, _, _, _, _) in layers]

    dloc, bsrc, cnt = _sc_prep(dst_p, src_p)
    h, t, p = _tc_dense0(x_p, W_in, b_in, wcat[0], bcat[0])

    for i, (_, b1, w2, b2, g, be) in enumerate(layers):
        pre2 = _sc_gather(t, jnp.reshape(p, (NP * H,)), dloc, bsrc)
        m2 = _tc_edge_mlp(pre2, w2, b2)
        agg2 = _sc_scatter_max(m2, dloc, cnt)
        agg = jnp.reshape(agg2, (NP, H))
        hp, ssum, ssq = _tc_stats(h, agg, first=(i == 0))
        if i < 2:
            h, t, p = _tc_apply(hp, ssum, ssq, g, be, wcat[i + 1], bcat[i + 1])
        else:
            z = _tc_final(hp, ssum, ssq, g, be, out_w1, out_b1, out_w2, out_b2)
    return z


# speedup vs baseline: 2.9620x; 1.2206x over previous
import functools

import jax
import jax.numpy as jnp
from jax import lax
from jax.experimental import pallas as pl
from jax.experimental.pallas import tpu as pltpu
from jax.experimental.pallas import tpu_sc as plsc

N = 50000
E = 800000
H = 64
D_LAT = 32

NW = 32
R = 1568
NP = NW * R
EP = 802816
CAP = 28672
NE2 = NW * CAP // 2
GCH = 64
SUP = 1024
SCH = 128

_f32 = jnp.float32
_i32 = jnp.int32


BLK = 3136


def _rowmask_at(blk, i):
    it = lax.broadcasted_iota(_i32, (blk, 1), 0) + i * blk
    return it < N


def _dense0_body(x_ref, win_ref, bin_ref, wcat_ref, bcat_ref, h_ref, t_ref, p_ref):
    i = pl.program_id(0)
    h = jnp.dot(x_ref[...], win_ref[...], preferred_element_type=_f32) + bin_ref[...]
    h = jnp.where(_rowmask_at(BLK, i), h, 0.0)
    h_ref[...] = h
    t = jnp.dot(h, wcat_ref[...], preferred_element_type=_f32) + bcat_ref[...]
    t_ref[...] = t
    p_ref[...] = t[:, H:]


def _tc_dense0(x_p, w_in, b_in, wcat, bcat):
    return pl.pallas_call(
        _dense0_body,
        grid=(NP // BLK,),
        in_specs=[
            pl.BlockSpec((BLK, 3), lambda i: (i, 0)),
            pl.BlockSpec((3, H), lambda i: (0, 0)),
            pl.BlockSpec((1, H), lambda i: (0, 0)),
            pl.BlockSpec((H, 2 * H), lambda i: (0, 0)),
            pl.BlockSpec((1, 2 * H), lambda i: (0, 0)),
        ],
        out_specs=(
            pl.BlockSpec((BLK, H), lambda i: (i, 0)),
            pl.BlockSpec((BLK, 2 * H), lambda i: (i, 0)),
            pl.BlockSpec((BLK, H), lambda i: (i, 0)),
        ),
        out_shape=(
            jax.ShapeDtypeStruct((NP, H), _f32),
            jax.ShapeDtypeStruct((NP, 2 * H), _f32),
            jax.ShapeDtypeStruct((NP, H), _f32),
        ),
    )(x_p, w_in, b_in.reshape(1, H), wcat, bcat)


def _edge_mlp_body(pre_ref, w2_ref, b2_ref, m_ref):
    a = jnp.maximum(pre_ref[:, :H], 0.0)
    b = jnp.maximum(pre_ref[:, H:], 0.0)
    ma = jnp.dot(a, w2_ref[...], preferred_element_type=_f32) + b2_ref[...]
    mb = jnp.dot(b, w2_ref[...], preferred_element_type=_f32) + b2_ref[...]
    m_ref[...] = jnp.concatenate([ma, mb], axis=1)


def _tc_edge_mlp(pre2, w2, b2):
    blk = 1024
    grid = NE2 // blk
    return pl.pallas_call(
        _edge_mlp_body,
        grid=(grid,),
        in_specs=[
            pl.BlockSpec((blk, 2 * H), lambda i: (i, 0)),
            pl.BlockSpec((H, H), lambda i: (0, 0)),
            pl.BlockSpec((1, H), lambda i: (0, 0)),
        ],
        out_specs=pl.BlockSpec((blk, 2 * H), lambda i: (i, 0)),
        out_shape=jax.ShapeDtypeStruct((NE2, 2 * H), _f32),
    )(pre2, w2, b2.reshape(1, H))


def _stats_body(first, h_ref, agg_ref, hp_ref, ssum_ref, ssq_ref):
    i = pl.program_id(0)
    aggc = jnp.where(jnp.isneginf(agg_ref[...]), 0.0, agg_ref[...])
    aggc = jnp.where(_rowmask_at(BLK, i), aggc, 0.0)
    hp = aggc if first else h_ref[...] + aggc
    hp_ref[...] = hp

    @pl.when(i == 0)
    def _():
        ssum_ref[...] = jnp.zeros((1, H), _f32)
        ssq_ref[...] = jnp.zeros((1, H), _f32)
    ssum_ref[...] += jnp.sum(hp, axis=0, keepdims=True)
    ssq_ref[...] += jnp.sum(hp * hp, axis=0, keepdims=True)


def _tc_stats(h, agg, first):
    return pl.pallas_call(
        functools.partial(_stats_body, first),
        grid=(NP // BLK,),
        in_specs=[
            pl.BlockSpec((BLK, H), lambda i: (i, 0)),
            pl.BlockSpec((BLK, H), lambda i: (i, 0)),
        ],
        out_specs=(
            pl.BlockSpec((BLK, H), lambda i: (i, 0)),
            pl.BlockSpec((1, H), lambda i: (0, 0)),
            pl.BlockSpec((1, H), lambda i: (0, 0)),
        ),
        out_shape=(
            jax.ShapeDtypeStruct((NP, H), _f32),
            jax.ShapeDtypeStruct((1, H), _f32),
            jax.ShapeDtypeStruct((1, H), _f32),
        ),
    )(h, agg)


def _hn_block(hp, ssum, ssq, g, be, i):
    mean = ssum / N
    var = ssq / N - mean * mean
    hn = g * (hp - mean) * lax.rsqrt(var + 1e-5) + be
    hn = jnp.maximum(hn, 0.0)
    return jnp.where(_rowmask_at(BLK, i), hn, 0.0)


def _apply_body(hp_ref, ssum_ref, ssq_ref, g_ref, be_ref,
                wcat_ref, bcat_ref, h_ref, t_ref, p_ref):
    i = pl.program_id(0)
    hn = _hn_block(hp_ref[...], ssum_ref[...], ssq_ref[...],
                   g_ref[...], be_ref[...], i)
    h_ref[...] = hn
    t = jnp.dot(hn, wcat_ref[...], preferred_element_type=_f32) + bcat_ref[...]
    t_ref[...] = t
    p_ref[...] = t[:, H:]


def _tc_apply(hp, ssum, ssq, g, be, wcat, bcat):
    return pl.pallas_call(
        _apply_body,
        grid=(NP // BLK,),
        in_specs=[
            pl.BlockSpec((BLK, H), lambda i: (i, 0)),
            pl.BlockSpec((1, H), lambda i: (0, 0)),
            pl.BlockSpec((1, H), lambda i: (0, 0)),
            pl.BlockSpec((1, H), lambda i: (0, 0)),
            pl.BlockSpec((1, H), lambda i: (0, 0)),
            pl.BlockSpec((H, 2 * H), lambda i: (0, 0)),
            pl.BlockSpec((1, 2 * H), lambda i: (0, 0)),
        ],
        out_specs=(
            pl.BlockSpec((BLK, H), lambda i: (i, 0)),
            pl.BlockSpec((BLK, 2 * H), lambda i: (i, 0)),
            pl.BlockSpec((BLK, H), lambda i: (i, 0)),
        ),
        out_shape=(
            jax.ShapeDtypeStruct((NP, H), _f32),
            jax.ShapeDtypeStruct((NP, 2 * H), _f32),
            jax.ShapeDtypeStruct((NP, H), _f32),
        ),
    )(hp, ssum, ssq, g.reshape(1, H), be.reshape(1, H), wcat, bcat)


def _final_sum_body(hp_ref, ssum_ref, ssq_ref, g_ref, be_ref, zsum_ref):
    i = pl.program_id(0)
    hn = _hn_block(hp_ref[...], ssum_ref[...], ssq_ref[...],
                   g_ref[...], be_ref[...], i)

    @pl.when(i == 0)
    def _():
        zsum_ref[...] = jnp.zeros((1, H), _f32)
    zsum_ref[...] += jnp.sum(hn, axis=0, keepdims=True)


def _head_body(zsum_ref, ow1_ref, ob1_ref, ow2_ref, ob2_ref, z_ref):
    z = zsum_ref[...] / N
    z = jnp.maximum(jnp.dot(z, ow1_ref[...], preferred_element_type=_f32)
                    + ob1_ref[...], 0.0)
    z_ref[...] = jnp.dot(z, ow2_ref[...], preferred_element_type=_f32) + ob2_ref[...]


def _tc_final(hp, ssum, ssq, g, be, ow1, ob1, ow2, ob2):
    zsum = pl.pallas_call(
        _final_sum_body,
        grid=(NP // BLK,),
        in_specs=[
            pl.BlockSpec((BLK, H), lambda i: (i, 0)),
            pl.BlockSpec((1, H), lambda i: (0, 0)),
            pl.BlockSpec((1, H), lambda i: (0, 0)),
            pl.BlockSpec((1, H), lambda i: (0, 0)),
            pl.BlockSpec((1, H), lambda i: (0, 0)),
        ],
        out_specs=pl.BlockSpec((1, H), lambda i: (0, 0)),
        out_shape=jax.ShapeDtypeStruct((1, H), _f32),
    )(hp, ssum, ssq, g.reshape(1, H), be.reshape(1, H))
    return pl.pallas_call(
        _head_body,
        out_shape=jax.ShapeDtypeStruct((1, D_LAT), _f32),
    )(zsum, ow1, ob1.reshape(1, H), ow2, ob2.reshape(1, D_LAT))


@functools.lru_cache(maxsize=None)
def _mesh():
    return plsc.VectorSubcoreMesh(core_axis_name="c", subcore_axis_name="s")


def _wid():
    return lax.axis_index("s") * 2 + lax.axis_index("c")


def _sc_prep(dst_p, src_p):
    @functools.partial(
        pl.kernel, mesh=_mesh(),
        compiler_params=pltpu.CompilerParams(needs_layout_passes=False),
        out_type=[
            jax.ShapeDtypeStruct((NW * CAP,), _i32),
            jax.ShapeDtypeStruct((NW * CAP,), _i32),
            jax.ShapeDtypeStruct((NW * 128,), _i32),
        ],
        scratch_types=[
            pltpu.VMEM((CAP,), _i32),
            pltpu.VMEM((CAP,), _i32),
            pltpu.VMEM((2048,), _i32),
            pltpu.VMEM((2048,), _i32),
            pltpu.VMEM((2048,), _i32),
            pltpu.VMEM((2048,), _i32),
            pltpu.VMEM((128,), _i32),
            pltpu.SemaphoreType.DMA,
            pltpu.SemaphoreType.DMA,
        ],
    )
    def k(dst_hbm, src_hbm, dloc_hbm, bsrc_hbm, cnt_hbm,
          dloc_v, bsrc_v, d0_v, s0_v, d1_v, s1_v, cnt_v, sm0, sm1):
        w = _wid()
        base_lo = w * R
        lanes = lax.iota(_i32, 16)
        db = (d0_v, d1_v)
        sb = (s0_v, s1_v)
        sems = (sm0, sm1)
        nch = EP // 2048

        def fill(i, _):
            dloc_v[pl.ds(i * 16, 16)] = jnp.full((16,), R, _i32)
            bsrc_v[pl.ds(i * 16, 16)] = i * 16 + lanes
            return 0
        lax.fori_loop(0, CAP // 16, fill, 0)

        for j in range(2):
            cb = pl.multiple_of(j * 2048, 2048)
            pltpu.async_copy(dst_hbm.at[pl.ds(cb, 2048)], db[j], sems[j])
            pltpu.async_copy(src_hbm.at[pl.ds(cb, 2048)], sb[j], sems[j])

        def outer(c, off):
            for b in range(2):
                cc = c * 2 + b
                cb = pl.multiple_of(cc * 2048, 2048)
                pltpu.make_async_copy(dst_hbm.at[pl.ds(cb, 2048)], db[b], sems[b]).wait()
                pltpu.make_async_copy(src_hbm.at[pl.ds(cb, 2048)], sb[b], sems[b]).wait()

                def group(gidx, off):
                    dl = db[b][pl.ds(gidx * 16, 16)] - base_lo
                    sv = sb[b][pl.ds(gidx * 16, 16)]
                    mask = (dl >= 0) & (dl < R)
                    cs = plsc.cumsum(mask.astype(_i32))
                    pos = jnp.minimum(off + cs - 1, CAP - 1)
                    plsc.store_scatter(dloc_v, [pos], dl, mask=mask)
                    plsc.store_scatter(bsrc_v, [pos], sv, mask=mask)
                    return off + jnp.max(cs, axis=0)
                off = lax.fori_loop(0, 128, group, off)

                nb = pl.multiple_of(jnp.minimum(cc + 2, nch - 1) * 2048, 2048)
                pltpu.async_copy(dst_hbm.at[pl.ds(nb, 2048)], db[b], sems[b])
                pltpu.async_copy(src_hbm.at[pl.ds(nb, 2048)], sb[b], sems[b])
            return off

        off = lax.fori_loop(0, nch // 2, outer, jnp.asarray(0, _i32))
        for j in range(2):
            pltpu.make_async_copy(dst_hbm.at[pl.ds(0, 2048)], db[j], sems[j]).wait()
            pltpu.make_async_copy(src_hbm.at[pl.ds(0, 2048)], sb[j], sems[j]).wait()
        off = jnp.minimum(off, CAP - 16)
        wb = pl.multiple_of(w * CAP, CAP)
        pltpu.sync_copy(dloc_v, dloc_hbm.at[pl.ds(wb, CAP)])
        pltpu.sync_copy(bsrc_v, bsrc_hbm.at[pl.ds(wb, CAP)])

        def cfill(i, _):
            cnt_v[pl.ds(i * 16, 16)] = jnp.full((16,), off, _i32)
            return 0
        lax.fori_loop(0, 8, cfill, 0)
        pltpu.sync_copy(cnt_v, cnt_hbm.at[pl.ds(pl.multiple_of(w * 128, 128), 128)])

    return k(dst_p, src_p)


def _sc_gather(t_tab, p_tab, dloc, bsrc):
    @functools.partial(
        pl.kernel, mesh=_mesh(),
        compiler_params=pltpu.CompilerParams(needs_layout_passes=False),
        out_type=jax.ShapeDtypeStruct((NE2, 2 * H), _f32),
        scratch_types=[
            pltpu.VMEM(((R + 1) * H,), _f32),
            pltpu.VMEM((SUP,), _i32),
            pltpu.VMEM((SUP,), _i32),
            pltpu.VMEM((GCH, 2 * H), _f32),
            pltpu.VMEM((GCH, 2 * H), _f32),
            pltpu.VMEM((GCH // 2, 2 * H), _f32),
            pltpu.VMEM((GCH // 2, 2 * H), _f32),
            pltpu.SemaphoreType.DMA,
            pltpu.SemaphoreType.DMA,
            pltpu.SemaphoreType.DMA,
            pltpu.SemaphoreType.DMA,
        ],
    )
    def k(t_hbm, p_hbm, dloc_hbm, bsrc_hbm, pre_hbm,
          p_v, dl_v, sr_v, q0_v, q1_v, pre0_v, pre1_v, g0, g1, w0, w1):
        w = _wid()
        wr = pl.multiple_of(w * R, R)
        lanes = lax.iota(_i32, 16)
        qb = (q0_v, q1_v)
        pb = (pre0_v, pre1_v)
        gs = (g0, g1)
        ws = (w0, w1)
        ninner = SUP // GCH

        pltpu.sync_copy(p_hbm.at[pl.ds(pl.multiple_of(w * (R * H), R * H), R * H)],
                        p_v.at[pl.ds(0, R * H)])

        def sup(c, _):
            sb = pl.multiple_of(w * CAP + c * SUP, SUP)
            pltpu.sync_copy(dloc_hbm.at[pl.ds(sb, SUP)], dl_v)
            pltpu.sync_copy(bsrc_hbm.at[pl.ds(sb, SUP)], sr_v)
            for j in range(2):
                pltpu.async_copy(
                    t_hbm.at[sr_v.at[pl.ds(j * GCH, GCH)]], qb[j], gs[j])
            for j in range(ninner):
                b = j & 1
                pltpu.make_async_copy(
                    t_hbm.at[sr_v.at[pl.ds(j * GCH, GCH)]], qb[b], gs[b]).wait()
                if j >= 2:
                    pltpu.make_async_copy(
                        pb[b], pre_hbm.at[pl.ds(0, GCH // 2)], ws[b]).wait()

                def grp(gi, _):
                    dl = dl_v[pl.ds(j * GCH + gi * 16, 16)]
                    for e in range(16):
                        d = jnp.max(jnp.where(lanes == e, dl, 0), axis=0)
                        erow = (gi * 16 + e) // 2
                        ecol = ((gi * 16 + e) % 2) * H
                        for q in range(H // 16):
                            a = p_v[pl.ds(d * H + q * 16, 16)]
                            b_ = qb[b][gi * 16 + e, pl.ds(q * 16, 16)]
                            pb[b][erow, pl.ds(ecol + q * 16, 16)] = a + b_
                    return 0
                lax.fori_loop(0, GCH // 16, grp, 0)

                ob = pl.multiple_of((sb + j * GCH) // 2, GCH // 2)
                pltpu.async_copy(pb[b], pre_hbm.at[pl.ds(ob, GCH // 2)], ws[b])
                if j + 2 < ninner:
                    pltpu.async_copy(
                        t_hbm.at[sr_v.at[pl.ds((j + 2) * GCH, GCH)]], qb[b], gs[b])
            for j in range(2):
                pltpu.make_async_copy(
                    pb[j], pre_hbm.at[pl.ds(0, GCH // 2)], ws[j]).wait()
            return 0

        lax.fori_loop(0, CAP // SUP, sup, 0)

    return k(t_tab, p_tab, dloc, bsrc)


def _sc_scatter_max(m2, dloc, cnt):
    @functools.partial(
        pl.kernel, mesh=_mesh(),
        compiler_params=pltpu.CompilerParams(needs_layout_passes=False),
        out_type=jax.ShapeDtypeStruct((NP // 2, 2 * H), _f32),
        scratch_types=[
            pltpu.VMEM((R // 2 + 1, 2 * H), _f32),
            pltpu.VMEM((SCH // 2, 2 * H), _f32),
            pltpu.VMEM((SCH // 2, 2 * H), _f32),
            pltpu.VMEM((SCH,), _i32),
            pltpu.VMEM((SCH,), _i32),
            pltpu.VMEM((128,), _i32),
            pltpu.SemaphoreType.DMA,
            pltpu.SemaphoreType.DMA,
        ],
    )
    def k(m_hbm, dloc_hbm, cnt_hbm, agg_hbm,
          acc_v, r0_v, r1_v, dl0_v, dl1_v, cnt_v, sm0, sm1):
        w = _wid()
        lanes = lax.iota(_i32, 16)
        neg = jnp.full((16,), -jnp.inf, _f32)
        rb = (r0_v, r1_v)
        dlb = (dl0_v, dl1_v)
        sems = (sm0, sm1)
        nmax = CAP // SCH

        def fill(i, _):
            for q in range(2 * H // 16):
                acc_v[i, pl.ds(q * 16, 16)] = neg
            return 0
        lax.fori_loop(0, R // 2 + 1, fill, 0)

        pltpu.sync_copy(cnt_hbm.at[pl.ds(pl.multiple_of(w * 128, 128), 128)], cnt_v)
        cnt = jnp.max(cnt_v[pl.ds(0, 16)], axis=0)
        nch2 = (cnt + (2 * SCH - 1)) // (2 * SCH)

        def start(cc, b):
            ci = jnp.minimum(cc, nmax - 1)
            cb = pl.multiple_of(w * CAP + ci * SCH, SCH)
            pltpu.async_copy(dloc_hbm.at[pl.ds(cb, SCH)], dlb[b], sems[b])
            pltpu.async_copy(
                m_hbm.at[pl.ds(pl.multiple_of(cb // 2, SCH // 2), SCH // 2)],
                rb[b], sems[b])

        def drain(b):
            pltpu.make_async_copy(
                dloc_hbm.at[pl.ds(0, SCH)], dlb[b], sems[b]).wait()
            pltpu.make_async_copy(
                m_hbm.at[pl.ds(0, SCH // 2)], rb[b], sems[b]).wait()

        for j in range(2):
            start(jnp.asarray(j, _i32), j)

        def outer(c, _):
            for b in range(2):
                cc = c * 2 + b
                drain(b)

                def group(gidx, _):
                    dl = dlb[b][pl.ds(gidx * 16, 16)]
                    for e in range(16):
                        jj = gidx * 16 + e
                        d = jnp.max(jnp.where(lanes == e, dl, 0), axis=0)
                        drow = d // 2
                        dcol = (d % 2) * H
                        for q in range(H // 16):
                            cur = acc_v[drow, pl.ds(dcol + q * 16, 16)]
                            val = rb[b][jj // 2, pl.ds((jj % 2) * H + q * 16, 16)]
                            acc_v[drow, pl.ds(dcol + q * 16, 16)] = jnp.maximum(cur, val)
                    return 0
                lax.fori_loop(0, SCH // 16, group, 0)
                start(cc + 2, b)
            return 0

        lax.fori_loop(0, nch2, outer, 0)
        for j in range(2):
            drain(j)
        pltpu.sync_copy(acc_v.at[pl.ds(0, R // 2)],
                        agg_hbm.at[pl.ds(pl.multiple_of(w * (R // 2), R // 2), R // 2)])

    return k(m2, dloc, cnt)


def kernel(x, edge_index, W_in, b_in,
           l0_w1, l0_b1, l0_w2, l0_b2, l0_g, l0_be,
           l1_w1, l1_b1, l1_w2, l1_b2, l1_g, l1_be,
           l2_w1, l2_b1, l2_w2, l2_b2, l2_g, l2_be,
           out_w1, out_b1, out_w2, out_b2):
    x_p = jnp.pad(x, ((0, NP - N), (0, 0)))
    pad_idx = N + (jnp.arange(EP - E, dtype=_i32) % (NP - N))
    dst_p = jnp.concatenate([edge_index[1], pad_idx])
    src_p = jnp.concatenate([edge_index[0], pad_idx % N])

    layers = [(l0_w1, l0_b1, l0_w2, l0_b2, l0_g, l0_be),
              (l1_w1, l1_b1, l1_w2, l1_b2, l1_g, l1_be),
              (l2_w1, l2_b1, l2_w2, l2_b2, l2_g, l2_be)]
    wcat = [jnp.concatenate([w1[H:], w1[:H] - w1[H:]], axis=1)
            for (w1, _, _, _, _, _) in layers]
    bcat = [jnp.concatenate([jnp.zeros((H,), _f32), b1]).reshape(1, 2 * H)
            for (_, b1, _, _, _, _) in layers]

    dloc, bsrc, cnt = _sc_prep(dst_p, src_p)
    h, t, p = _tc_dense0(x_p, W_in, b_in, wcat[0], bcat[0])

    for i, (_, b1, w2, b2, g, be) in enumerate(layers):
        pre2 = _sc_gather(t, jnp.reshape(p, (NP * H,)), dloc, bsrc)
        m2 = _tc_edge_mlp(pre2, w2, b2)
        agg2 = _sc_scatter_max(m2, dloc, cnt)
        agg = jnp.reshape(agg2, (NP, H))
        hp, ssum, ssq = _tc_stats(h, agg, first=(i == 0))
        if i < 2:
            h, t, p = _tc_apply(hp, ssum, ssq, g, be, wcat[i + 1], bcat[i + 1])
        else:
            z = _tc_final(hp, ssum, ssq, g, be, out_w1, out_b1, out_w2, out_b2)
    return z


# speedup vs baseline: 2.9976x; 1.0120x over previous
import functools

import jax
import jax.numpy as jnp
from jax import lax
from jax.experimental import pallas as pl
from jax.experimental.pallas import tpu as pltpu
from jax.experimental.pallas import tpu_sc as plsc

N = 50000
E = 800000
H = 64
D_LAT = 32

NW = 32
R = 1568
NP = NW * R
EP = 802816
CAP = 28672
NE2 = NW * CAP // 2
GCH = 64
SUP = 1024
SCH = 128

_f32 = jnp.float32
_i32 = jnp.int32


BLK = 3136


def _rowmask_at(blk, i):
    it = lax.broadcasted_iota(_i32, (blk, 1), 0) + i * blk
    return it < N


def _dense0_body(x_ref, win_ref, bin_ref, wcat_ref, bcat_ref, h_ref, t_ref, p_ref):
    i = pl.program_id(0)
    h = jnp.dot(x_ref[...], win_ref[...], preferred_element_type=_f32) + bin_ref[...]
    h = jnp.where(_rowmask_at(BLK, i), h, 0.0)
    h_ref[...] = h
    t = jnp.dot(h, wcat_ref[...], preferred_element_type=_f32) + bcat_ref[...]
    t_ref[...] = t
    p_ref[...] = t[:, H:]


def _tc_dense0(x_p, w_in, b_in, wcat, bcat):
    return pl.pallas_call(
        _dense0_body,
        grid=(NP // BLK,),
        in_specs=[
            pl.BlockSpec((BLK, 3), lambda i: (i, 0)),
            pl.BlockSpec((3, H), lambda i: (0, 0)),
            pl.BlockSpec((1, H), lambda i: (0, 0)),
            pl.BlockSpec((H, 2 * H), lambda i: (0, 0)),
            pl.BlockSpec((1, 2 * H), lambda i: (0, 0)),
        ],
        out_specs=(
            pl.BlockSpec((BLK, H), lambda i: (i, 0)),
            pl.BlockSpec((BLK, 2 * H), lambda i: (i, 0)),
            pl.BlockSpec((BLK, H), lambda i: (i, 0)),
        ),
        out_shape=(
            jax.ShapeDtypeStruct((NP, H), _f32),
            jax.ShapeDtypeStruct((NP, 2 * H), _f32),
            jax.ShapeDtypeStruct((NP, H), _f32),
        ),
    )(x_p, w_in, b_in.reshape(1, H), wcat, bcat)


def _edge_mlp_body(pre_ref, w2_ref, b2_ref, m_ref):
    a = jnp.maximum(pre_ref[:, :H], 0.0)
    b = jnp.maximum(pre_ref[:, H:], 0.0)
    ma = jnp.dot(a, w2_ref[...], preferred_element_type=_f32) + b2_ref[...]
    mb = jnp.dot(b, w2_ref[...], preferred_element_type=_f32) + b2_ref[...]
    m_ref[...] = jnp.concatenate([ma, mb], axis=1)


def _tc_edge_mlp(pre2, w2, b2):
    blk = 1024
    grid = NE2 // blk
    return pl.pallas_call(
        _edge_mlp_body,
        grid=(grid,),
        in_specs=[
            pl.BlockSpec((blk, 2 * H), lambda i: (i, 0)),
            pl.BlockSpec((H, H), lambda i: (0, 0)),
            pl.BlockSpec((1, H), lambda i: (0, 0)),
        ],
        out_specs=pl.BlockSpec((blk, 2 * H), lambda i: (i, 0)),
        out_shape=jax.ShapeDtypeStruct((NE2, 2 * H), _f32),
    )(pre2, w2, b2.reshape(1, H))


def _stats_body(first, h_ref, agg_ref, hp_ref, ssum_ref, ssq_ref):
    i = pl.program_id(0)
    aggc = jnp.where(jnp.isneginf(agg_ref[...]), 0.0, agg_ref[...])
    aggc = jnp.where(_rowmask_at(BLK, i), aggc, 0.0)
    hp = aggc if first else h_ref[...] + aggc
    hp_ref[...] = hp

    @pl.when(i == 0)
    def _():
        ssum_ref[...] = jnp.zeros((1, H), _f32)
        ssq_ref[...] = jnp.zeros((1, H), _f32)
    ssum_ref[...] += jnp.sum(hp, axis=0, keepdims=True)
    ssq_ref[...] += jnp.sum(hp * hp, axis=0, keepdims=True)


def _tc_stats(h, agg, first):
    return pl.pallas_call(
        functools.partial(_stats_body, first),
        grid=(NP // BLK,),
        in_specs=[
            pl.BlockSpec((BLK, H), lambda i: (i, 0)),
            pl.BlockSpec((BLK, H), lambda i: (i, 0)),
        ],
        out_specs=(
            pl.BlockSpec((BLK, H), lambda i: (i, 0)),
            pl.BlockSpec((1, H), lambda i: (0, 0)),
            pl.BlockSpec((1, H), lambda i: (0, 0)),
        ),
        out_shape=(
            jax.ShapeDtypeStruct((NP, H), _f32),
            jax.ShapeDtypeStruct((1, H), _f32),
            jax.ShapeDtypeStruct((1, H), _f32),
        ),
    )(h, agg)


def _hn_block(hp, ssum, ssq, g, be, i):
    mean = ssum / N
    var = ssq / N - mean * mean
    hn = g * (hp - mean) * lax.rsqrt(var + 1e-5) + be
    hn = jnp.maximum(hn, 0.0)
    return jnp.where(_rowmask_at(BLK, i), hn, 0.0)


def _apply_body(hp_ref, ssum_ref, ssq_ref, g_ref, be_ref,
                wcat_ref, bcat_ref, h_ref, t_ref, p_ref):
    i = pl.program_id(0)
    hn = _hn_block(hp_ref[...], ssum_ref[...], ssq_ref[...],
                   g_ref[...], be_ref[...], i)
    h_ref[...] = hn
    t = jnp.dot(hn, wcat_ref[...], preferred_element_type=_f32) + bcat_ref[...]
    t_ref[...] = t
    p_ref[...] = t[:, H:]


def _tc_apply(hp, ssum, ssq, g, be, wcat, bcat):
    return pl.pallas_call(
        _apply_body,
        grid=(NP // BLK,),
        in_specs=[
            pl.BlockSpec((BLK, H), lambda i: (i, 0)),
            pl.BlockSpec((1, H), lambda i: (0, 0)),
            pl.BlockSpec((1, H), lambda i: (0, 0)),
            pl.BlockSpec((1, H), lambda i: (0, 0)),
            pl.BlockSpec((1, H), lambda i: (0, 0)),
            pl.BlockSpec((H, 2 * H), lambda i: (0, 0)),
            pl.BlockSpec((1, 2 * H), lambda i: (0, 0)),
        ],
        out_specs=(
            pl.BlockSpec((BLK, H), lambda i: (i, 0)),
            pl.BlockSpec((BLK, 2 * H), lambda i: (i, 0)),
            pl.BlockSpec((BLK, H), lambda i: (i, 0)),
        ),
        out_shape=(
            jax.ShapeDtypeStruct((NP, H), _f32),
            jax.ShapeDtypeStruct((NP, 2 * H), _f32),
            jax.ShapeDtypeStruct((NP, H), _f32),
        ),
    )(hp, ssum, ssq, g.reshape(1, H), be.reshape(1, H), wcat, bcat)


def _final_sum_body(hp_ref, ssum_ref, ssq_ref, g_ref, be_ref, zsum_ref):
    i = pl.program_id(0)
    hn = _hn_block(hp_ref[...], ssum_ref[...], ssq_ref[...],
                   g_ref[...], be_ref[...], i)

    @pl.when(i == 0)
    def _():
        zsum_ref[...] = jnp.zeros((1, H), _f32)
    zsum_ref[...] += jnp.sum(hn, axis=0, keepdims=True)


def _head_body(zsum_ref, ow1_ref, ob1_ref, ow2_ref, ob2_ref, z_ref):
    z = zsum_ref[...] / N
    z = jnp.maximum(jnp.dot(z, ow1_ref[...], preferred_element_type=_f32)
                    + ob1_ref[...], 0.0)
    z_ref[...] = jnp.dot(z, ow2_ref[...], preferred_element_type=_f32) + ob2_ref[...]


def _tc_final(hp, ssum, ssq, g, be, ow1, ob1, ow2, ob2):
    zsum = pl.pallas_call(
        _final_sum_body,
        grid=(NP // BLK,),
        in_specs=[
            pl.BlockSpec((BLK, H), lambda i: (i, 0)),
            pl.BlockSpec((1, H), lambda i: (0, 0)),
            pl.BlockSpec((1, H), lambda i: (0, 0)),
            pl.BlockSpec((1, H), lambda i: (0, 0)),
            pl.BlockSpec((1, H), lambda i: (0, 0)),
        ],
        out_specs=pl.BlockSpec((1, H), lambda i: (0, 0)),
        out_shape=jax.ShapeDtypeStruct((1, H), _f32),
    )(hp, ssum, ssq, g.reshape(1, H), be.reshape(1, H))
    return pl.pallas_call(
        _head_body,
        out_shape=jax.ShapeDtypeStruct((1, D_LAT), _f32),
    )(zsum, ow1, ob1.reshape(1, H), ow2, ob2.reshape(1, D_LAT))


@functools.lru_cache(maxsize=None)
def _mesh():
    return plsc.VectorSubcoreMesh(core_axis_name="c", subcore_axis_name="s")


def _wid():
    return lax.axis_index("s") * 2 + lax.axis_index("c")


def _sc_prep(dst_p, src_p):
    @functools.partial(
        pl.kernel, mesh=_mesh(),
        compiler_params=pltpu.CompilerParams(needs_layout_passes=False),
        out_type=[
            jax.ShapeDtypeStruct((NW * CAP,), _i32),
            jax.ShapeDtypeStruct((NW * CAP,), _i32),
            jax.ShapeDtypeStruct((NW * 128,), _i32),
        ],
        scratch_types=[
            pltpu.VMEM((CAP,), _i32),
            pltpu.VMEM((CAP,), _i32),
            pltpu.VMEM((2048,), _i32),
            pltpu.VMEM((2048,), _i32),
            pltpu.VMEM((2048,), _i32),
            pltpu.VMEM((2048,), _i32),
            pltpu.VMEM((128,), _i32),
            pltpu.SemaphoreType.DMA,
            pltpu.SemaphoreType.DMA,
        ],
    )
    def k(dst_hbm, src_hbm, dloc_hbm, bsrc_hbm, cnt_hbm,
          dloc_v, bsrc_v, d0_v, s0_v, d1_v, s1_v, cnt_v, sm0, sm1):
        w = _wid()
        base_lo = w * R
        lanes = lax.iota(_i32, 16)
        db = (d0_v, d1_v)
        sb = (s0_v, s1_v)
        sems = (sm0, sm1)
        nch = EP // 2048

        def fill(i, _):
            dloc_v[pl.ds(i * 16, 16)] = jnp.full((16,), R, _i32)
            bsrc_v[pl.ds(i * 16, 16)] = i * 16 + lanes
            return 0
        lax.fori_loop(0, CAP // 16, fill, 0)

        for j in range(2):
            cb = pl.multiple_of(j * 2048, 2048)
            pltpu.async_copy(dst_hbm.at[pl.ds(cb, 2048)], db[j], sems[j])
            pltpu.async_copy(src_hbm.at[pl.ds(cb, 2048)], sb[j], sems[j])

        def outer(c, off):
            for b in range(2):
                cc = c * 2 + b
                cb = pl.multiple_of(cc * 2048, 2048)
                pltpu.make_async_copy(dst_hbm.at[pl.ds(cb, 2048)], db[b], sems[b]).wait()
                pltpu.make_async_copy(src_hbm.at[pl.ds(cb, 2048)], sb[b], sems[b]).wait()

                def group(gidx, off):
                    dl = db[b][pl.ds(gidx * 16, 16)] - base_lo
                    sv = sb[b][pl.ds(gidx * 16, 16)]
                    mask = (dl >= 0) & (dl < R)
                    cs = plsc.cumsum(mask.astype(_i32))
                    pos = jnp.minimum(off + cs - 1, CAP - 1)
                    plsc.store_scatter(dloc_v, [pos], dl, mask=mask)
                    plsc.store_scatter(bsrc_v, [pos], sv, mask=mask)
                    return off + jnp.max(cs, axis=0)
                off = lax.fori_loop(0, 128, group, off)

                nb = pl.multiple_of(jnp.minimum(cc + 2, nch - 1) * 2048, 2048)
                pltpu.async_copy(dst_hbm.at[pl.ds(nb, 2048)], db[b], sems[b])
                pltpu.async_copy(src_hbm.at[pl.ds(nb, 2048)], sb[b], sems[b])
            return off

        off = lax.fori_loop(0, nch // 2, outer, jnp.asarray(0, _i32))
        for j in range(2):
            pltpu.make_async_copy(dst_hbm.at[pl.ds(0, 2048)], db[j], sems[j]).wait()
            pltpu.make_async_copy(src_hbm.at[pl.ds(0, 2048)], sb[j], sems[j]).wait()
        off = jnp.minimum(off, CAP - 16)
        wb = pl.multiple_of(w * CAP, CAP)
        pltpu.sync_copy(dloc_v, dloc_hbm.at[pl.ds(wb, CAP)])
        pltpu.sync_copy(bsrc_v, bsrc_hbm.at[pl.ds(wb, CAP)])

        def cfill(i, _):
            cnt_v[pl.ds(i * 16, 16)] = jnp.full((16,), off, _i32)
            return 0
        lax.fori_loop(0, 8, cfill, 0)
        pltpu.sync_copy(cnt_v, cnt_hbm.at[pl.ds(pl.multiple_of(w * 128, 128), 128)])

    return k(dst_p, src_p)


def _sc_gather(t_tab, p_tab, dloc, bsrc, cnt):
    @functools.partial(
        pl.kernel, mesh=_mesh(),
        compiler_params=pltpu.CompilerParams(needs_layout_passes=False),
        out_type=jax.ShapeDtypeStruct((NE2, 2 * H), _f32),
        scratch_types=[
            pltpu.VMEM(((R + 1) * H,), _f32),
            pltpu.VMEM((SUP,), _i32),
            pltpu.VMEM((SUP,), _i32),
            pltpu.VMEM((GCH, 2 * H), _f32),
            pltpu.VMEM((GCH, 2 * H), _f32),
            pltpu.VMEM((GCH // 2, 2 * H), _f32),
            pltpu.VMEM((GCH // 2, 2 * H), _f32),
            pltpu.VMEM((128,), _i32),
            pltpu.SemaphoreType.DMA,
            pltpu.SemaphoreType.DMA,
            pltpu.SemaphoreType.DMA,
            pltpu.SemaphoreType.DMA,
        ],
    )
    def k(t_hbm, p_hbm, dloc_hbm, bsrc_hbm, cnt_hbm, pre_hbm,
          p_v, dl_v, sr_v, q0_v, q1_v, pre0_v, pre1_v, cnt_v, g0, g1, w0, w1):
        w = _wid()
        wr = pl.multiple_of(w * R, R)
        lanes = lax.iota(_i32, 16)
        qb = (q0_v, q1_v)
        pb = (pre0_v, pre1_v)
        gs = (g0, g1)
        ws = (w0, w1)
        ninner = SUP // GCH

        pltpu.sync_copy(p_hbm.at[pl.ds(pl.multiple_of(w * (R * H), R * H), R * H)],
                        p_v.at[pl.ds(0, R * H)])
        pltpu.sync_copy(cnt_hbm.at[pl.ds(pl.multiple_of(w * 128, 128), 128)], cnt_v)
        cnt = jnp.max(cnt_v[pl.ds(0, 16)], axis=0)
        nsup = (cnt + SUP - 1) // SUP

        def sup(c, _):
            sb = pl.multiple_of(w * CAP + c * SUP, SUP)
            pltpu.sync_copy(dloc_hbm.at[pl.ds(sb, SUP)], dl_v)
            pltpu.sync_copy(bsrc_hbm.at[pl.ds(sb, SUP)], sr_v)
            for j in range(2):
                pltpu.async_copy(
                    t_hbm.at[sr_v.at[pl.ds(j * GCH, GCH)]], qb[j], gs[j])
            for j in range(ninner):
                b = j & 1
                pltpu.make_async_copy(
                    t_hbm.at[sr_v.at[pl.ds(j * GCH, GCH)]], qb[b], gs[b]).wait()
                if j >= 2:
                    pltpu.make_async_copy(
                        pb[b], pre_hbm.at[pl.ds(0, GCH // 2)], ws[b]).wait()

                def grp(gi, _):
                    dl = dl_v[pl.ds(j * GCH + gi * 16, 16)]
                    for e in range(16):
                        base = dl[lanes * 0 + e] * H + lanes
                        erow = (gi * 16 + e) // 2
                        ecol = ((gi * 16 + e) % 2) * H
                        for q in range(H // 16):
                            a = plsc.load_gather(p_v, [base + q * 16])
                            b_ = qb[b][gi * 16 + e, pl.ds(q * 16, 16)]
                            pb[b][erow, pl.ds(ecol + q * 16, 16)] = a + b_
                    return 0
                lax.fori_loop(0, GCH // 16, grp, 0)

                ob = pl.multiple_of((sb + j * GCH) // 2, GCH // 2)
                pltpu.async_copy(pb[b], pre_hbm.at[pl.ds(ob, GCH // 2)], ws[b])
                if j + 2 < ninner:
                    pltpu.async_copy(
                        t_hbm.at[sr_v.at[pl.ds((j + 2) * GCH, GCH)]], qb[b], gs[b])
            for j in range(2):
                pltpu.make_async_copy(
                    pb[j], pre_hbm.at[pl.ds(0, GCH // 2)], ws[j]).wait()
            return 0

        lax.fori_loop(0, nsup, sup, 0)

    return k(t_tab, p_tab, dloc, bsrc, cnt)


def _sc_scatter_max(m2, dloc, cnt):
    @functools.partial(
        pl.kernel, mesh=_mesh(),
        compiler_params=pltpu.CompilerParams(needs_layout_passes=False),
        out_type=jax.ShapeDtypeStruct((NP * H,), _f32),
        scratch_types=[
            pltpu.VMEM(((R + 1) * H,), _f32),
            pltpu.VMEM((SCH // 2, 2 * H), _f32),
            pltpu.VMEM((SCH // 2, 2 * H), _f32),
            pltpu.VMEM((SCH,), _i32),
            pltpu.VMEM((SCH,), _i32),
            pltpu.VMEM((128,), _i32),
            pltpu.SemaphoreType.DMA,
            pltpu.SemaphoreType.DMA,
        ],
    )
    def k(m_hbm, dloc_hbm, cnt_hbm, agg_hbm,
          acc_v, r0_v, r1_v, dl0_v, dl1_v, cnt_v, sm0, sm1):
        w = _wid()
        lanes = lax.iota(_i32, 16)
        neg = jnp.full((16,), -jnp.inf, _f32)
        rb = (r0_v, r1_v)
        dlb = (dl0_v, dl1_v)
        sems = (sm0, sm1)
        nmax = CAP // SCH

        def fill(i, _):
            for q in range(8):
                acc_v[pl.ds((i * 8 + q) * 16, 16)] = neg
            return 0
        lax.fori_loop(0, (R + 1) * H // 128, fill, 0)

        pltpu.sync_copy(cnt_hbm.at[pl.ds(pl.multiple_of(w * 128, 128), 128)], cnt_v)
        cnt = jnp.max(cnt_v[pl.ds(0, 16)], axis=0)
        nch2 = (cnt + (2 * SCH - 1)) // (2 * SCH)

        def start(cc, b):
            ci = jnp.minimum(cc, nmax - 1)
            cb = pl.multiple_of(w * CAP + ci * SCH, SCH)
            pltpu.async_copy(dloc_hbm.at[pl.ds(cb, SCH)], dlb[b], sems[b])
            pltpu.async_copy(
                m_hbm.at[pl.ds(pl.multiple_of(cb // 2, SCH // 2), SCH // 2)],
                rb[b], sems[b])

        def drain(b):
            pltpu.make_async_copy(
                dloc_hbm.at[pl.ds(0, SCH)], dlb[b], sems[b]).wait()
            pltpu.make_async_copy(
                m_hbm.at[pl.ds(0, SCH // 2)], rb[b], sems[b]).wait()

        for j in range(2):
            start(jnp.asarray(j, _i32), j)

        def outer(c, _):
            for b in range(2):
                cc = c * 2 + b
                drain(b)

                def group(gidx, _):
                    dl = dlb[b][pl.ds(gidx * 16, 16)]
                    for e in range(16):
                        jj = gidx * 16 + e
                        base = dl[lanes * 0 + e] * H + lanes
                        for q in range(H // 16):
                            idx = base + q * 16
                            cur = plsc.load_gather(acc_v, [idx])
                            val = rb[b][jj // 2, pl.ds((jj % 2) * H + q * 16, 16)]
                            plsc.store_scatter(acc_v, [idx], jnp.maximum(cur, val))
                    return 0
                lax.fori_loop(0, SCH // 16, group, 0)
                start(cc + 2, b)
            return 0

        lax.fori_loop(0, nch2, outer, 0)
        for j in range(2):
            drain(j)
        pltpu.sync_copy(acc_v.at[pl.ds(0, R * H)],
                        agg_hbm.at[pl.ds(pl.multiple_of(w * (R * H), R * H), R * H)])

    return k(m2, dloc, cnt)


def kernel(x, edge_index, W_in, b_in,
           l0_w1, l0_b1, l0_w2, l0_b2, l0_g, l0_be,
           l1_w1, l1_b1, l1_w2, l1_b2, l1_g, l1_be,
           l2_w1, l2_b1, l2_w2, l2_b2, l2_g, l2_be,
           out_w1, out_b1, out_w2, out_b2):
    x_p = jnp.pad(x, ((0, NP - N), (0, 0)))
    pad_idx = N + (jnp.arange(EP - E, dtype=_i32) % (NP - N))
    dst_p = jnp.concatenate([edge_index[1], pad_idx])
    src_p = jnp.concatenate([edge_index[0], pad_idx % N])

    layers = [(l0_w1, l0_b1, l0_w2, l0_b2, l0_g, l0_be),
              (l1_w1, l1_b1, l1_w2, l1_b2, l1_g, l1_be),
              (l2_w1, l2_b1, l2_w2, l2_b2, l2_g, l2_be)]
    wcat = [jnp.concatenate([w1[H:], w1[:H] - w1[H:]], axis=1)
            for (w1, _, _, _, _, _) in layers]
    bcat = [jnp.concatenate([jnp.zeros((H,), _f32), b1]).reshape(1, 2 * H)
            for (_, b1, _, _, _, _) in layers]

    dloc, bsrc, cnt = _sc_prep(dst_p, src_p)
    h, t, p = _tc_dense0(x_p, W_in, b_in, wcat[0], bcat[0])

    for i, (_, b1, w2, b2, g, be) in enumerate(layers):
        pre2 = _sc_gather(t, jnp.reshape(p, (NP * H,)), dloc, bsrc, cnt)
        m2 = _tc_edge_mlp(pre2, w2, b2)
        agg_flat = _sc_scatter_max(m2, dloc, cnt)
        agg = jnp.reshape(agg_flat, (NP, H))
        hp, ssum, ssq = _tc_stats(h, agg, first=(i == 0))
        if i < 2:
            h, t, p = _tc_apply(hp, ssum, ssq, g, be, wcat[i + 1], bcat[i + 1])
        else:
            z = _tc_final(hp, ssum, ssq, g, be, out_w1, out_b1, out_w2, out_b2)
    return z


# speedup vs baseline: 2.9999x; 1.0007x over previous
import functools

import jax
import jax.numpy as jnp
from jax import lax
from jax.experimental import pallas as pl
from jax.experimental.pallas import tpu as pltpu
from jax.experimental.pallas import tpu_sc as plsc

N = 50000
E = 800000
H = 64
D_LAT = 32

NW = 32
R = 1568
NP = NW * R
EP = 802816
CAP = 28672
NE2 = NW * CAP // 2
GCH = 64
SUP = 1024
SCH = 128

_f32 = jnp.float32
_i32 = jnp.int32


BLK = 3136


def _rowmask_at(blk, i):
    it = lax.broadcasted_iota(_i32, (blk, 1), 0) + i * blk
    return it < N


def _dense0_body(x_ref, win_ref, bin_ref, wcat_ref, bcat_ref, h_ref, t_ref, p_ref):
    i = pl.program_id(0)
    h = jnp.dot(x_ref[...], win_ref[...], preferred_element_type=_f32) + bin_ref[...]
    h = jnp.where(_rowmask_at(BLK, i), h, 0.0)
    h_ref[...] = h
    t = jnp.dot(h, wcat_ref[...], preferred_element_type=_f32) + bcat_ref[...]
    t_ref[...] = t
    p_ref[...] = t[:, H:]


def _tc_dense0(x_p, w_in, b_in, wcat, bcat):
    return pl.pallas_call(
        _dense0_body,
        grid=(NP // BLK,),
        in_specs=[
            pl.BlockSpec((BLK, 3), lambda i: (i, 0)),
            pl.BlockSpec((3, H), lambda i: (0, 0)),
            pl.BlockSpec((1, H), lambda i: (0, 0)),
            pl.BlockSpec((H, 2 * H), lambda i: (0, 0)),
            pl.BlockSpec((1, 2 * H), lambda i: (0, 0)),
        ],
        out_specs=(
            pl.BlockSpec((BLK, H), lambda i: (i, 0)),
            pl.BlockSpec((BLK, 2 * H), lambda i: (i, 0)),
            pl.BlockSpec((BLK, H), lambda i: (i, 0)),
        ),
        out_shape=(
            jax.ShapeDtypeStruct((NP, H), _f32),
            jax.ShapeDtypeStruct((NP, 2 * H), _f32),
            jax.ShapeDtypeStruct((NP, H), _f32),
        ),
    )(x_p, w_in, b_in.reshape(1, H), wcat, bcat)


def _edge_mlp_body(pre_ref, w2_ref, b2_ref, m_ref):
    a = jnp.maximum(pre_ref[:, :H], 0.0)
    b = jnp.maximum(pre_ref[:, H:], 0.0)
    ma = jnp.dot(a, w2_ref[...], preferred_element_type=_f32) + b2_ref[...]
    mb = jnp.dot(b, w2_ref[...], preferred_element_type=_f32) + b2_ref[...]
    m_ref[...] = jnp.concatenate([ma, mb], axis=1)


def _tc_edge_mlp(pre2, w2, b2):
    blk = 1024
    grid = NE2 // blk
    return pl.pallas_call(
        _edge_mlp_body,
        grid=(grid,),
        in_specs=[
            pl.BlockSpec((blk, 2 * H), lambda i: (i, 0)),
            pl.BlockSpec((H, H), lambda i: (0, 0)),
            pl.BlockSpec((1, H), lambda i: (0, 0)),
        ],
        out_specs=pl.BlockSpec((blk, 2 * H), lambda i: (i, 0)),
        out_shape=jax.ShapeDtypeStruct((NE2, 2 * H), _f32),
    )(pre2, w2, b2.reshape(1, H))


def _stats_body(first, h_ref, agg_ref, hp_ref, ssum_ref, ssq_ref):
    i = pl.program_id(0)
    aggc = jnp.where(jnp.isneginf(agg_ref[...]), 0.0, agg_ref[...])
    aggc = jnp.where(_rowmask_at(BLK, i), aggc, 0.0)
    hp = aggc if first else h_ref[...] + aggc
    hp_ref[...] = hp

    @pl.when(i == 0)
    def _():
        ssum_ref[...] = jnp.zeros((1, H), _f32)
        ssq_ref[...] = jnp.zeros((1, H), _f32)
    ssum_ref[...] += jnp.sum(hp, axis=0, keepdims=True)
    ssq_ref[...] += jnp.sum(hp * hp, axis=0, keepdims=True)


def _tc_stats(h, agg, first):
    return pl.pallas_call(
        functools.partial(_stats_body, first),
        grid=(NP // BLK,),
        in_specs=[
            pl.BlockSpec((BLK, H), lambda i: (i, 0)),
            pl.BlockSpec((BLK, H), lambda i: (i, 0)),
        ],
        out_specs=(
            pl.BlockSpec((BLK, H), lambda i: (i, 0)),
            pl.BlockSpec((1, H), lambda i: (0, 0)),
            pl.BlockSpec((1, H), lambda i: (0, 0)),
        ),
        out_shape=(
            jax.ShapeDtypeStruct((NP, H), _f32),
            jax.ShapeDtypeStruct((1, H), _f32),
            jax.ShapeDtypeStruct((1, H), _f32),
        ),
    )(h, agg)


def _hn_block(hp, ssum, ssq, g, be, i):
    mean = ssum / N
    var = ssq / N - mean * mean
    hn = g * (hp - mean) * lax.rsqrt(var + 1e-5) + be
    hn = jnp.maximum(hn, 0.0)
    return jnp.where(_rowmask_at(BLK, i), hn, 0.0)


def _apply_body(hp_ref, ssum_ref, ssq_ref, g_ref, be_ref,
                wcat_ref, bcat_ref, h_ref, t_ref, p_ref):
    i = pl.program_id(0)
    hn = _hn_block(hp_ref[...], ssum_ref[...], ssq_ref[...],
                   g_ref[...], be_ref[...], i)
    h_ref[...] = hn
    t = jnp.dot(hn, wcat_ref[...], preferred_element_type=_f32) + bcat_ref[...]
    t_ref[...] = t
    p_ref[...] = t[:, H:]


def _tc_apply(hp, ssum, ssq, g, be, wcat, bcat):
    return pl.pallas_call(
        _apply_body,
        grid=(NP // BLK,),
        in_specs=[
            pl.BlockSpec((BLK, H), lambda i: (i, 0)),
            pl.BlockSpec((1, H), lambda i: (0, 0)),
            pl.BlockSpec((1, H), lambda i: (0, 0)),
            pl.BlockSpec((1, H), lambda i: (0, 0)),
            pl.BlockSpec((1, H), lambda i: (0, 0)),
            pl.BlockSpec((H, 2 * H), lambda i: (0, 0)),
            pl.BlockSpec((1, 2 * H), lambda i: (0, 0)),
        ],
        out_specs=(
            pl.BlockSpec((BLK, H), lambda i: (i, 0)),
            pl.BlockSpec((BLK, 2 * H), lambda i: (i, 0)),
            pl.BlockSpec((BLK, H), lambda i: (i, 0)),
        ),
        out_shape=(
            jax.ShapeDtypeStruct((NP, H), _f32),
            jax.ShapeDtypeStruct((NP, 2 * H), _f32),
            jax.ShapeDtypeStruct((NP, H), _f32),
        ),
    )(hp, ssum, ssq, g.reshape(1, H), be.reshape(1, H), wcat, bcat)


def _final_sum_body(hp_ref, ssum_ref, ssq_ref, g_ref, be_ref, zsum_ref):
    i = pl.program_id(0)
    hn = _hn_block(hp_ref[...], ssum_ref[...], ssq_ref[...],
                   g_ref[...], be_ref[...], i)

    @pl.when(i == 0)
    def _():
        zsum_ref[...] = jnp.zeros((1, H), _f32)
    zsum_ref[...] += jnp.sum(hn, axis=0, keepdims=True)


def _head_body(zsum_ref, ow1_ref, ob1_ref, ow2_ref, ob2_ref, z_ref):
    z = zsum_ref[...] / N
    z = jnp.maximum(jnp.dot(z, ow1_ref[...], preferred_element_type=_f32)
                    + ob1_ref[...], 0.0)
    z_ref[...] = jnp.dot(z, ow2_ref[...], preferred_element_type=_f32) + ob2_ref[...]


def _tc_final(hp, ssum, ssq, g, be, ow1, ob1, ow2, ob2):
    zsum = pl.pallas_call(
        _final_sum_body,
        grid=(NP // BLK,),
        in_specs=[
            pl.BlockSpec((BLK, H), lambda i: (i, 0)),
            pl.BlockSpec((1, H), lambda i: (0, 0)),
            pl.BlockSpec((1, H), lambda i: (0, 0)),
            pl.BlockSpec((1, H), lambda i: (0, 0)),
            pl.BlockSpec((1, H), lambda i: (0, 0)),
        ],
        out_specs=pl.BlockSpec((1, H), lambda i: (0, 0)),
        out_shape=jax.ShapeDtypeStruct((1, H), _f32),
    )(hp, ssum, ssq, g.reshape(1, H), be.reshape(1, H))
    return pl.pallas_call(
        _head_body,
        out_shape=jax.ShapeDtypeStruct((1, D_LAT), _f32),
    )(zsum, ow1, ob1.reshape(1, H), ow2, ob2.reshape(1, D_LAT))


@functools.lru_cache(maxsize=None)
def _mesh():
    return plsc.VectorSubcoreMesh(core_axis_name="c", subcore_axis_name="s")


def _wid():
    return lax.axis_index("s") * 2 + lax.axis_index("c")


def _sc_prep(dst_p, src_p):
    @functools.partial(
        pl.kernel, mesh=_mesh(),
        compiler_params=pltpu.CompilerParams(needs_layout_passes=False),
        out_type=[
            jax.ShapeDtypeStruct((NW * CAP,), _i32),
            jax.ShapeDtypeStruct((NW * CAP,), _i32),
            jax.ShapeDtypeStruct((NW * 128,), _i32),
        ],
        scratch_types=[
            pltpu.VMEM((CAP,), _i32),
            pltpu.VMEM((CAP,), _i32),
            pltpu.VMEM((2048,), _i32),
            pltpu.VMEM((2048,), _i32),
            pltpu.VMEM((2048,), _i32),
            pltpu.VMEM((2048,), _i32),
            pltpu.VMEM((128,), _i32),
            pltpu.SemaphoreType.DMA,
            pltpu.SemaphoreType.DMA,
        ],
    )
    def k(dst_hbm, src_hbm, dloc_hbm, bsrc_hbm, cnt_hbm,
          dloc_v, bsrc_v, d0_v, s0_v, d1_v, s1_v, cnt_v, sm0, sm1):
        w = _wid()
        base_lo = w * R
        lanes = lax.iota(_i32, 16)
        db = (d0_v, d1_v)
        sb = (s0_v, s1_v)
        sems = (sm0, sm1)
        nch = EP // 2048

        def fill(i, _):
            dloc_v[pl.ds(i * 16, 16)] = jnp.full((16,), R, _i32)
            bsrc_v[pl.ds(i * 16, 16)] = i * 16 + lanes
            return 0
        lax.fori_loop(0, CAP // 16, fill, 0)

        for j in range(2):
            cb = pl.multiple_of(j * 2048, 2048)
            pltpu.async_copy(dst_hbm.at[pl.ds(cb, 2048)], db[j], sems[j])
            pltpu.async_copy(src_hbm.at[pl.ds(cb, 2048)], sb[j], sems[j])

        def outer(c, off):
            for b in range(2):
                cc = c * 2 + b
                cb = pl.multiple_of(cc * 2048, 2048)
                pltpu.make_async_copy(dst_hbm.at[pl.ds(cb, 2048)], db[b], sems[b]).wait()
                pltpu.make_async_copy(src_hbm.at[pl.ds(cb, 2048)], sb[b], sems[b]).wait()

                def group(gidx, off):
                    dl = db[b][pl.ds(gidx * 16, 16)] - base_lo
                    sv = sb[b][pl.ds(gidx * 16, 16)]
                    mask = (dl >= 0) & (dl < R)
                    cs = plsc.cumsum(mask.astype(_i32))
                    pos = jnp.minimum(off + cs - 1, CAP - 1)
                    plsc.store_scatter(dloc_v, [pos], dl, mask=mask)
                    plsc.store_scatter(bsrc_v, [pos], sv, mask=mask)
                    return off + cs[lanes * 0 + 15]
                off = lax.fori_loop(0, 128, group, off)

                nb = pl.multiple_of(jnp.minimum(cc + 2, nch - 1) * 2048, 2048)
                pltpu.async_copy(dst_hbm.at[pl.ds(nb, 2048)], db[b], sems[b])
                pltpu.async_copy(src_hbm.at[pl.ds(nb, 2048)], sb[b], sems[b])
            return off

        off_vec = lax.fori_loop(0, nch // 2, outer, jnp.zeros((16,), _i32))
        off = jnp.max(off_vec, axis=0)
        for j in range(2):
            pltpu.make_async_copy(dst_hbm.at[pl.ds(0, 2048)], db[j], sems[j]).wait()
            pltpu.make_async_copy(src_hbm.at[pl.ds(0, 2048)], sb[j], sems[j]).wait()
        off = jnp.minimum(off, CAP - 16)
        wb = pl.multiple_of(w * CAP, CAP)
        pltpu.sync_copy(dloc_v, dloc_hbm.at[pl.ds(wb, CAP)])
        pltpu.sync_copy(bsrc_v, bsrc_hbm.at[pl.ds(wb, CAP)])

        def cfill(i, _):
            cnt_v[pl.ds(i * 16, 16)] = jnp.full((16,), off, _i32)
            return 0
        lax.fori_loop(0, 8, cfill, 0)
        pltpu.sync_copy(cnt_v, cnt_hbm.at[pl.ds(pl.multiple_of(w * 128, 128), 128)])

    return k(dst_p, src_p)


def _sc_gather(t_tab, p_tab, dloc, bsrc, cnt):
    @functools.partial(
        pl.kernel, mesh=_mesh(),
        compiler_params=pltpu.CompilerParams(needs_layout_passes=False),
        out_type=jax.ShapeDtypeStruct((NE2, 2 * H), _f32),
        scratch_types=[
            pltpu.VMEM(((R + 1) * H,), _f32),
            pltpu.VMEM((SUP,), _i32),
            pltpu.VMEM((SUP,), _i32),
            pltpu.VMEM((GCH, 2 * H), _f32),
            pltpu.VMEM((GCH, 2 * H), _f32),
            pltpu.VMEM((GCH // 2, 2 * H), _f32),
            pltpu.VMEM((GCH // 2, 2 * H), _f32),
            pltpu.VMEM((128,), _i32),
            pltpu.SemaphoreType.DMA,
            pltpu.SemaphoreType.DMA,
            pltpu.SemaphoreType.DMA,
            pltpu.SemaphoreType.DMA,
        ],
    )
    def k(t_hbm, p_hbm, dloc_hbm, bsrc_hbm, cnt_hbm, pre_hbm,
          p_v, dl_v, sr_v, q0_v, q1_v, pre0_v, pre1_v, cnt_v, g0, g1, w0, w1):
        w = _wid()
        wr = pl.multiple_of(w * R, R)
        lanes = lax.iota(_i32, 16)
        qb = (q0_v, q1_v)
        pb = (pre0_v, pre1_v)
        gs = (g0, g1)
        ws = (w0, w1)
        ninner = SUP // GCH

        pltpu.sync_copy(p_hbm.at[pl.ds(pl.multiple_of(w * (R * H), R * H), R * H)],
                        p_v.at[pl.ds(0, R * H)])
        pltpu.sync_copy(cnt_hbm.at[pl.ds(pl.multiple_of(w * 128, 128), 128)], cnt_v)
        cnt = jnp.max(cnt_v[pl.ds(0, 16)], axis=0)
        nsup = (cnt + SUP - 1) // SUP

        def sup(c, _):
            sb = pl.multiple_of(w * CAP + c * SUP, SUP)
            pltpu.sync_copy(dloc_hbm.at[pl.ds(sb, SUP)], dl_v)
            pltpu.sync_copy(bsrc_hbm.at[pl.ds(sb, SUP)], sr_v)
            for j in range(2):
                pltpu.async_copy(
                    t_hbm.at[sr_v.at[pl.ds(j * GCH, GCH)]], qb[j], gs[j])
            for j in range(ninner):
                b = j & 1
                pltpu.make_async_copy(
                    t_hbm.at[sr_v.at[pl.ds(j * GCH, GCH)]], qb[b], gs[b]).wait()
                if j >= 2:
                    pltpu.make_async_copy(
                        pb[b], pre_hbm.at[pl.ds(0, GCH // 2)], ws[b]).wait()

                def grp(gi, _):
                    dl = dl_v[pl.ds(j * GCH + gi * 16, 16)]
                    for e in range(16):
                        base = dl[lanes * 0 + e] * H + lanes
                        erow = (gi * 16 + e) // 2
                        ecol = ((gi * 16 + e) % 2) * H
                        for q in range(H // 16):
                            a = plsc.load_gather(p_v, [base + q * 16])
                            b_ = qb[b][gi * 16 + e, pl.ds(q * 16, 16)]
                            pb[b][erow, pl.ds(ecol + q * 16, 16)] = a + b_
                    return 0
                lax.fori_loop(0, GCH // 16, grp, 0)

                ob = pl.multiple_of((sb + j * GCH) // 2, GCH // 2)
                pltpu.async_copy(pb[b], pre_hbm.at[pl.ds(ob, GCH // 2)], ws[b])
                if j + 2 < ninner:
                    pltpu.async_copy(
                        t_hbm.at[sr_v.at[pl.ds((j + 2) * GCH, GCH)]], qb[b], gs[b])
            for j in range(2):
                pltpu.make_async_copy(
                    pb[j], pre_hbm.at[pl.ds(0, GCH // 2)], ws[j]).wait()
            return 0

        lax.fori_loop(0, nsup, sup, 0)

    return k(t_tab, p_tab, dloc, bsrc, cnt)


def _sc_scatter_max(m2, dloc, cnt):
    @functools.partial(
        pl.kernel, mesh=_mesh(),
        compiler_params=pltpu.CompilerParams(needs_layout_passes=False),
        out_type=jax.ShapeDtypeStruct((NP * H,), _f32),
        scratch_types=[
            pltpu.VMEM(((R + 1) * H,), _f32),
            pltpu.VMEM((SCH // 2, 2 * H), _f32),
            pltpu.VMEM((SCH // 2, 2 * H), _f32),
            pltpu.VMEM((SCH,), _i32),
            pltpu.VMEM((SCH,), _i32),
            pltpu.VMEM((128,), _i32),
            pltpu.SemaphoreType.DMA,
            pltpu.SemaphoreType.DMA,
        ],
    )
    def k(m_hbm, dloc_hbm, cnt_hbm, agg_hbm,
          acc_v, r0_v, r1_v, dl0_v, dl1_v, cnt_v, sm0, sm1):
        w = _wid()
        lanes = lax.iota(_i32, 16)
        neg = jnp.full((16,), -jnp.inf, _f32)
        rb = (r0_v, r1_v)
        dlb = (dl0_v, dl1_v)
        sems = (sm0, sm1)
        nmax = CAP // SCH

        def fill(i, _):
            for q in range(8):
                acc_v[pl.ds((i * 8 + q) * 16, 16)] = neg
            return 0
        lax.fori_loop(0, (R + 1) * H // 128, fill, 0)

        pltpu.sync_copy(cnt_hbm.at[pl.ds(pl.multiple_of(w * 128, 128), 128)], cnt_v)
        cnt = jnp.max(cnt_v[pl.ds(0, 16)], axis=0)
        nch2 = (cnt + (2 * SCH - 1)) // (2 * SCH)

        def start(cc, b):
            ci = jnp.minimum(cc, nmax - 1)
            cb = pl.multiple_of(w * CAP + ci * SCH, SCH)
            pltpu.async_copy(dloc_hbm.at[pl.ds(cb, SCH)], dlb[b], sems[b])
            pltpu.async_copy(
                m_hbm.at[pl.ds(pl.multiple_of(cb // 2, SCH // 2), SCH // 2)],
                rb[b], sems[b])

        def drain(b):
            pltpu.make_async_copy(
                dloc_hbm.at[pl.ds(0, SCH)], dlb[b], sems[b]).wait()
            pltpu.make_async_copy(
                m_hbm.at[pl.ds(0, SCH // 2)], rb[b], sems[b]).wait()

        for j in range(2):
            start(jnp.asarray(j, _i32), j)

        def outer(c, _):
            for b in range(2):
                cc = c * 2 + b
                drain(b)

                def group(gidx, _):
                    dl = dlb[b][pl.ds(gidx * 16, 16)]
                    for e in range(16):
                        jj = gidx * 16 + e
                        base = dl[lanes * 0 + e] * H + lanes
                        for q in range(H // 16):
                            idx = base + q * 16
                            cur = plsc.load_gather(acc_v, [idx])
                            val = rb[b][jj // 2, pl.ds((jj % 2) * H + q * 16, 16)]
                            plsc.store_scatter(acc_v, [idx], jnp.maximum(cur, val))
                    return 0
                lax.fori_loop(0, SCH // 16, group, 0)
                start(cc + 2, b)
            return 0

        lax.fori_loop(0, nch2, outer, 0)
        for j in range(2):
            drain(j)
        pltpu.sync_copy(acc_v.at[pl.ds(0, R * H)],
                        agg_hbm.at[pl.ds(pl.multiple_of(w * (R * H), R * H), R * H)])

    return k(m2, dloc, cnt)


def kernel(x, edge_index, W_in, b_in,
           l0_w1, l0_b1, l0_w2, l0_b2, l0_g, l0_be,
           l1_w1, l1_b1, l1_w2, l1_b2, l1_g, l1_be,
           l2_w1, l2_b1, l2_w2, l2_b2, l2_g, l2_be,
           out_w1, out_b1, out_w2, out_b2):
    x_p = jnp.pad(x, ((0, NP - N), (0, 0)))
    pad_idx = N + (jnp.arange(EP - E, dtype=_i32) % (NP - N))
    dst_p = jnp.concatenate([edge_index[1], pad_idx])
    src_p = jnp.concatenate([edge_index[0], pad_idx % N])

    layers = [(l0_w1, l0_b1, l0_w2, l0_b2, l0_g, l0_be),
              (l1_w1, l1_b1, l1_w2, l1_b2, l1_g, l1_be),
              (l2_w1, l2_b1, l2_w2, l2_b2, l2_g, l2_be)]
    wcat = [jnp.concatenate([w1[H:], w1[:H] - w1[H:]], axis=1)
            for (w1, _, _, _, _, _) in layers]
    bcat = [jnp.concatenate([jnp.zeros((H,), _f32), b1]).reshape(1, 2 * H)
            for (_, b1, _, _, _, _) in layers]

    dloc, bsrc, cnt = _sc_prep(dst_p, src_p)
    h, t, p = _tc_dense0(x_p, W_in, b_in, wcat[0], bcat[0])

    for i, (_, b1, w2, b2, g, be) in enumerate(layers):
        pre2 = _sc_gather(t, jnp.reshape(p, (NP * H,)), dloc, bsrc, cnt)
        m2 = _tc_edge_mlp(pre2, w2, b2)
        agg_flat = _sc_scatter_max(m2, dloc, cnt)
        agg = jnp.reshape(agg_flat, (NP, H))
        hp, ssum, ssq = _tc_stats(h, agg, first=(i == 0))
        if i < 2:
            h, t, p = _tc_apply(hp, ssum, ssq, g, be, wcat[i + 1], bcat[i + 1])
        else:
            z = _tc_final(hp, ssum, ssq, g, be, out_w1, out_b1, out_w2, out_b2)
    return z
